# baseline (speedup 1.0000x reference)
import functools

import jax
import jax.numpy as jnp
import numpy as np
from jax.experimental import pallas as pl
from jax.experimental.pallas import tpu as pltpu

_BF = jnp.bfloat16
_F32 = jnp.float32


def _rup(x, m):
    return ((x + m - 1) // m) * m


def _tile(m, target, align=8):
    t = min(target, m)
    t -= t % align
    while t >= align:
        if m % t == 0:
            return t
        t -= align
    return m


def _interp_mat(out_size, in_size):
    if out_size == 1 or in_size == 1:
        m = np.zeros((out_size, in_size), np.float32)
        m[:, 0] = 1.0
        return m
    src = np.arange(out_size, dtype=np.float64) * (in_size - 1) / (out_size - 1)
    i0 = np.clip(np.floor(src).astype(np.int64), 0, in_size - 1)
    i1 = np.clip(i0 + 1, 0, in_size - 1)
    w1 = (src - i0).astype(np.float32)
    w0 = 1.0 - w1
    m = np.zeros((out_size, in_size), np.float32)
    m[np.arange(out_size), i0] += w0
    m[np.arange(out_size), i1] += w1
    return m


def _cparams():
    return pltpu.CompilerParams(
        dimension_semantics=("parallel",),
        vmem_limit_bytes=64 * 1024 * 1024,
    )


def _kron_eye(w2d, blocks):
    k, n = w2d.shape
    mask = np.kron(np.eye(blocks, dtype=np.float32), np.ones((k, n), np.float32))
    return (jnp.asarray(mask)
            * jnp.tile(w2d.astype(_F32), (blocks, blocks))).astype(_BF)


def _fold(w, scale):
    wf = w.astype(_F32)
    if scale is not None:
        wf = wf * scale[None, None, None, :]
    return wf


def _btile(bias, blocks):
    return jnp.tile(bias.astype(_F32), blocks).reshape(1, -1)


def _wsel3(wf, wp, wo, stride, dil, kp):
    cin, cout = wf.shape[2], wf.shape[3]
    masks = np.zeros((3, wp * cin, wo * cout), np.float32)
    cols = np.arange(wo)
    for kw in range(3):
        msel = np.zeros((wp, wo), np.float32)
        msel[cols * stride + kw * dil, cols] = 1.0
        masks[kw] = np.kron(msel, np.ones((cin, cout), np.float32))
    tiled = jnp.tile(wf.astype(_F32), (1, 1, wp, wo))
    wb = jnp.sum(jnp.asarray(masks)[None] * tiled, axis=1)
    return jnp.pad(wb, ((0, 0), (0, kp - wp * cin), (0, 0))).astype(_BF)


def _s2conv_body(x_ref, w_ref, b_ref, o_ref, *, ho, kp):
    nb = o_ref.shape[0]
    xs = x_ref[...]
    acc = None
    for kh in range(3):
        if kh == 0:
            a = xs[:, 0:ho, 0:kp]
        elif kh == 1:
            a = xs[:, 0:ho, kp:2 * kp]
        else:
            a = xs[:, 1:ho + 1, 0:kp]
        d = jnp.dot(a.reshape(nb * ho, kp), w_ref[kh],
                    preferred_element_type=_F32)
        acc = d if acc is None else acc + d
    acc = jnp.maximum(acc + b_ref[...], 0.0)
    o_ref[...] = acc.reshape(nb, ho, acc.shape[-1]).astype(o_ref.dtype)


def _s2conv(x3, w, scale, bias, wi, cin, cout):
    n, h, _ = x3.shape
    ho, wo = h // 2, wi // 2
    hp, wp = h + 2, wi + 2
    wpc = wp * cin
    kp = _rup(wpc, 128)
    xp = jnp.pad(x3, ((0, 0), (1, 1), (cin, kp - wpc + cin)))
    xp = xp.reshape(n, hp // 2, 2 * kp)
    wbig = _wsel3(_fold(w, scale), wp, wo, 2, 1, kp)
    bt = _btile(bias, wo)
    nb = min(max(128 // ho, 1), n)
    while n % nb:
        nb -= 1
    return pl.pallas_call(
        functools.partial(_s2conv_body, ho=ho, kp=kp),
        out_shape=jax.ShapeDtypeStruct((n, ho, wo * cout), _BF),
        grid=(n // nb,),
        in_specs=[pl.BlockSpec((nb, hp // 2, 2 * kp), lambda i: (i, 0, 0)),
                  pl.BlockSpec((3, kp, wo * cout), lambda i: (0, 0, 0)),
                  pl.BlockSpec((1, wo * cout), lambda i: (0, 0))],
        out_specs=pl.BlockSpec((nb, ho, wo * cout), lambda i: (i, 0, 0)),
        compiler_params=_cparams(),
    )(xp, wbig, bt)


def _s1pad(x3, wi, c):
    wpc = (wi + 2) * c
    kp = _rup(wpc, 128)
    return jnp.pad(x3, ((0, 0), (1, 1), (c, kp - wpc + c))), kp


def _dec_body(*refs, n_in, ho, kps, chain):
    x_refs = refs[:n_in]
    w_refs = refs[n_in:2 * n_in]
    b_ref = refs[2 * n_in]
    extra = refs[2 * n_in + 1:]
    nb = extra[-1].shape[0]
    acc = None
    for j in range(n_in):
        xs = x_refs[j][...]
        for kh in range(3):
            a = xs[:, kh:kh + ho, :].reshape(nb * ho, kps[j])
            d = jnp.dot(a, w_refs[j][kh], preferred_element_type=_F32)
            acc = d if acc is None else acc + d
    acc = jnp.maximum(acc + b_ref[...], 0.0)
    if chain:
        wc_ref, bc_ref, o_ref = extra
        acc2 = jnp.dot(acc.astype(_BF), wc_ref[...],
                       preferred_element_type=_F32) + bc_ref[...]
        o_ref[...] = acc2.reshape(nb, ho, acc2.shape[-1]).astype(o_ref.dtype)
    else:
        o_ref = extra[0]
        o_ref[...] = acc.reshape(nb, ho, acc.shape[-1]).astype(o_ref.dtype)


def _dec_conv(x3_list, cins, wf, bias, wi, cout, chain_w=None, chain_b=None):
    n, ho = x3_list[0].shape[0], x3_list[0].shape[1]
    xps, kps, wbigs = [], [], []
    off = 0
    for x3, cin in zip(x3_list, cins):
        xp, kp = _s1pad(x3, wi, cin)
        wfj = wf[:, :, off:off + cin, :]
        off += cin
        wb = _wsel3(wfj, wi + 2, wi, 1, 1, kp)
        xps.append(xp)
        kps.append(kp)
        wbigs.append(wb)
    bt = _btile(bias, wi)
    n_out = wi * cout
    chain = chain_w is not None
    if chain:
        ncls = chain_w.shape[1]
        wc = _kron_eye(chain_w, wi)
        bc = _btile(chain_b, wi)
        n_out = wi * ncls
    nb = min(max(128 // ho, 1), n)
    while n % nb:
        nb -= 1
    in_specs = (
        [pl.BlockSpec((nb, ho + 2, kp), lambda i: (i, 0, 0)) for kp in kps]
        + [pl.BlockSpec((3, kp, wi * cout), lambda i: (0, 0, 0)) for kp in kps]
        + [pl.BlockSpec((1, wi * cout), lambda i: (0, 0))]
    )
    ops = list(xps) + wbigs + [bt]
    if chain:
        in_specs += [pl.BlockSpec((wi * cout, n_out), lambda i: (0, 0)),
                     pl.BlockSpec((1, n_out), lambda i: (0, 0))]
        ops += [wc, bc]
    return pl.pallas_call(
        functools.partial(_dec_body, n_in=len(x3_list), ho=ho,
                          kps=tuple(kps), chain=chain),
        out_shape=jax.ShapeDtypeStruct((n, ho, n_out), _BF),
        grid=(n // nb,),
        in_specs=in_specs,
        out_specs=pl.BlockSpec((nb, ho, n_out), lambda i: (i, 0, 0)),
        compiler_params=_cparams(),
    )(*ops)


def _flat1_body(x_ref, w_ref, b_ref, o_ref):
    nb, ho, kp = x_ref.shape
    a = x_ref[...].reshape(nb * ho, kp)
    acc = jnp.maximum(jnp.dot(a, w_ref[...], preferred_element_type=_F32)
                      + b_ref[...], 0.0)
    o_ref[...] = acc.reshape(nb, ho, acc.shape[-1]).astype(o_ref.dtype)


def _flat1(x3, w2d, bias, wi):
    n, ho, _ = x3.shape
    wk = _kron_eye(w2d, wi)
    bt = _btile(bias, wi)
    n_out = wk.shape[1]
    nb = min(max(256 // ho, 1), n)
    while n % nb:
        nb -= 1
    return pl.pallas_call(
        _flat1_body,
        out_shape=jax.ShapeDtypeStruct((n, ho, n_out), _BF),
        grid=(n // nb,),
        in_specs=[pl.BlockSpec((nb, ho, x3.shape[2]), lambda i: (i, 0, 0)),
                  pl.BlockSpec((wk.shape[0], n_out), lambda i: (0, 0)),
                  pl.BlockSpec((1, n_out), lambda i: (0, 0))],
        out_specs=pl.BlockSpec((nb, ho, n_out), lambda i: (i, 0, 0)),
        compiler_params=_cparams(),
    )(x3, wk, bt)


def _aspp_body(h_ref, hp6_ref, w0_ref, w2_ref, w3_ref, wb1_ref, wp_ref,
               j0_ref, j1_ref, j2_ref, j3_ref, j4_ref, k8_ref,
               p2_ref, c8_ref, e2_ref,
               c0_ref, c1_ref, c2_ref, c3_ref, cp_ref, cj_ref, o_ref):
    nb, sf, lanes = o_ref.shape
    h = h_ref[...]
    b0 = jnp.maximum(jnp.dot(h, w0_ref[...], preferred_element_type=_F32)
                     + c0_ref[...], 0.0).astype(_BF)
    b2 = jnp.maximum(jnp.dot(h, w2_ref[...], preferred_element_type=_F32)
                     + c2_ref[...], 0.0).astype(_BF)
    b3 = jnp.maximum(jnp.dot(h, w3_ref[...], preferred_element_type=_F32)
                     + c3_ref[...], 0.0).astype(_BF)
    hp = hp6_ref[...]
    b1 = None
    for kh in range(3):
        a = hp[:, 6 * kh:6 * kh + sf, :].reshape(nb * sf, hp.shape[-1])
        d = jnp.dot(a, wb1_ref[kh], preferred_element_type=_F32)
        b1 = d if b1 is None else b1 + d
    b1 = jnp.maximum(b1 + c1_ref[...], 0.0).astype(_BF)
    acc = jnp.dot(b0, j0_ref[...], preferred_element_type=_F32)
    acc = acc + jnp.dot(b1, j1_ref[...], preferred_element_type=_F32)
    acc = acc + jnp.dot(b2, j2_ref[...], preferred_element_type=_F32)
    acc = acc + jnp.dot(b3, j3_ref[...], preferred_element_type=_F32)
    pr = jnp.dot(p2_ref[...], h, preferred_element_type=_F32)
    pm = jnp.dot(pr.astype(_BF), c8_ref[...], preferred_element_type=_F32)
    b4 = jnp.maximum(jnp.dot(pm.astype(_BF), wp_ref[...],
                             preferred_element_type=_F32) + cp_ref[...], 0.0)
    c4 = jnp.dot(b4.astype(_BF), j4_ref[...], preferred_element_type=_F32)
    c4t = jnp.dot(c4.astype(_BF), k8_ref[...], preferred_element_type=_F32)
    acc = acc + jnp.dot(e2_ref[...], c4t.astype(_BF),
                        preferred_element_type=_F32)
    acc = jnp.maximum(acc + cj_ref[...], 0.0)
    o_ref[...] = acc.reshape(nb, sf, lanes).astype(o_ref.dtype)


def _aspp(h4, w0, w1, w2, w3, wp, wj, biases, sf, cm, co):
    n = h4.shape[0]
    hflat = h4.reshape(n * sf, sf * cm)
    hp6 = jnp.pad(h4, ((0, 0), (6, 6), (6 * cm, 6 * cm)))
    g = 2 if n % 2 == 0 else 1
    nb = n // g
    wb1 = _wsel3(_fold(w1, None), sf + 12, sf, 1, 6, hp6.shape[2])
    k8 = np.zeros((co * sf, co * sf), np.float32)
    for wi_ in range(sf):
        k8[0:co, wi_ * co:(wi_ + 1) * co] = np.eye(co)
    p2 = np.kron(np.eye(n, dtype=np.float32), np.full((1, sf), 1.0 / sf))
    c8 = np.kron(np.full((sf, 1), 1.0 / sf, np.float32), np.eye(cm))
    e2 = np.kron(np.eye(n, dtype=np.float32), np.ones((sf, 1), np.float32))
    c0, c1, c2, c3 = [_btile(b, sf) for b in biases[:4]]
    cp = jnp.pad(biases[4].astype(_F32).reshape(1, -1),
                 ((0, 0), (0, co * sf - co)))
    cj = _btile(biases[5], sf)
    jpads = [_kron_eye(w, sf) for w in wj[:4]] + [
        jnp.pad(wj[4].astype(_F32), ((0, co * sf - wj[4].shape[0]),
                                     (0, co * sf - wj[4].shape[1]))).astype(_BF)]
    wpp = jnp.pad(wp.astype(_F32), ((0, 0), (0, co * sf - co))).astype(_BF)
    lanes = sf * co
    in_specs = [
        pl.BlockSpec((nb * sf, sf * cm), lambda i: (i, 0)),
        pl.BlockSpec((nb, sf + 12, hp6.shape[2]), lambda i: (i, 0, 0)),
        pl.BlockSpec((sf * cm, lanes), lambda i: (0, 0)),
        pl.BlockSpec((sf * cm, lanes), lambda i: (0, 0)),
        pl.BlockSpec((sf * cm, lanes), lambda i: (0, 0)),
        pl.BlockSpec((3, hp6.shape[2], lanes), lambda i: (0, 0, 0)),
        pl.BlockSpec((cm, lanes), lambda i: (0, 0)),
    ] + [pl.BlockSpec((lanes, lanes), lambda i: (0, 0))] * 6 + [
        pl.BlockSpec((n, nb * sf), lambda i: (0, i)),
        pl.BlockSpec((sf * cm, cm), lambda i: (0, 0)),
        pl.BlockSpec((nb * sf, n), lambda i: (i, 0)),
    ] + [pl.BlockSpec((1, lanes), lambda i: (0, 0))] * 6
    return pl.pallas_call(
        _aspp_body,
        out_shape=jax.ShapeDtypeStruct((n, sf, lanes), _BF),
        grid=(g,),
        in_specs=in_specs,
        out_specs=pl.BlockSpec((nb, sf, lanes), lambda i: (i, 0, 0)),
        compiler_params=_cparams(),
    )(hflat, hp6,
      _kron_eye(w0, sf), _kron_eye(w2, sf), _kron_eye(w3, sf), wb1, wpp,
      *jpads, jnp.asarray(k8).astype(_BF),
      jnp.asarray(p2).astype(_BF), jnp.asarray(c8).astype(_BF),
      jnp.asarray(e2).astype(_BF),
      c0, c1, c2, c3, cp, cj)


def _up_body(x_ref, ww_ref, rh_ref, o_ref):
    nb, ho, lanes = o_ref.shape
    sf = x_ref.shape[1]
    xm = jnp.dot(x_ref[...].reshape(nb * sf, x_ref.shape[2]), ww_ref[...],
                 preferred_element_type=_F32)
    hu = jnp.dot(rh_ref[...], xm.astype(_BF), preferred_element_type=_F32)
    o_ref[...] = hu.reshape(nb, ho, lanes).astype(o_ref.dtype)


def _up832(x3, sf, sd, co):
    n = x3.shape[0]
    r1 = _interp_mat(sd, sf)
    ww = np.einsum('ow,ij->wioj', r1, np.eye(co, dtype=np.float32))
    ww = jnp.asarray(ww.reshape(sf * co, sd * co)).astype(_BF)
    bigrh = jnp.asarray(np.kron(np.eye(n, dtype=np.float32), r1)).astype(_BF)
    g = 2 if n % 2 == 0 else 1
    nb = n // g
    return pl.pallas_call(
        _up_body,
        out_shape=jax.ShapeDtypeStruct((n, sd, sd * co), _BF),
        grid=(g,),
        in_specs=[pl.BlockSpec((nb, sf, sf * co), lambda i: (i, 0, 0)),
                  pl.BlockSpec((sf * co, sd * co), lambda i: (0, 0)),
                  pl.BlockSpec((nb * sd, nb * sf), lambda i: (i, i))],
        out_specs=pl.BlockSpec((nb, sd, sd * co), lambda i: (i, 0, 0)),
        compiler_params=_cparams(),
    )(x3, ww, bigrh)


def _mm_body(a_ref, b_ref, o_ref):
    o_ref[...] = jnp.dot(a_ref[...], b_ref[...],
                         preferred_element_type=_F32).astype(o_ref.dtype)


def _mmT_body(a_ref, b_ref, o_ref):
    o_ref[...] = jax.lax.dot_general(
        a_ref[...], b_ref[...], (((0,), (0,)), ((), ())),
        preferred_element_type=_F32).astype(o_ref.dtype)


def _mmT(at, b, tile_m, out_dtype):
    k, m = at.shape
    n = b.shape[1]
    tm = _tile(m, tile_m, align=128)
    return pl.pallas_call(
        _mmT_body,
        out_shape=jax.ShapeDtypeStruct((m, n), out_dtype),
        grid=(m // tm,),
        in_specs=[pl.BlockSpec((k, tm), lambda i: (0, i)),
                  pl.BlockSpec((k, n), lambda i: (0, 0))],
        out_specs=pl.BlockSpec((tm, n), lambda i: (i, 0)),
        compiler_params=_cparams(),
    )(at.astype(_BF), b.astype(_BF))


def _col_mm(a, b, tile_n, out_dtype):
    m, k = a.shape
    n = b.shape[1]
    tn = _tile(n, tile_n, align=128)
    return pl.pallas_call(
        _mm_body,
        out_shape=jax.ShapeDtypeStruct((m, n), out_dtype),
        grid=(n // tn,),
        in_specs=[pl.BlockSpec((m, k), lambda j: (0, 0)),
                  pl.BlockSpec((k, tn), lambda j: (0, j))],
        out_specs=pl.BlockSpec((m, tn), lambda j: (0, j)),
        compiler_params=_cparams(),
    )(a.astype(_BF), b.astype(_BF))


def kernel(stem1_w, stem1_scale, stem1_bias, stem2_w, stem2_scale, stem2_bias,
           layer3_w, layer3_scale, layer3_bias, layer4_w, layer4_scale,
           layer4_bias, aspp0_w, aspp0_scale, aspp0_bias, aspp1_w, aspp1_scale,
           aspp1_bias, aspp2_w, aspp2_scale, aspp2_bias, aspp3_w, aspp3_scale,
           aspp3_bias, aspp_pool_w, aspp_pool_scale, aspp_pool_bias,
           aspp_proj_w, aspp_proj_scale, aspp_proj_bias, dec_low_w,
           dec_low_scale, dec_low_bias, dec_conv1_w, dec_conv1_scale,
           dec_conv1_bias, dec_conv2_w, dec_conv2_scale, dec_conv2_bias,
           classifier_w, classifier_b, x):
    n, _, s, _ = x.shape
    xh = jnp.transpose(x, (0, 2, 3, 1)).astype(_BF).reshape(n, s, s * 3)

    h1 = _s2conv(xh, stem1_w, stem1_scale, stem1_bias, s, 3, 8)
    h2 = _s2conv(h1, stem2_w, stem2_scale, stem2_bias, s // 2, 8, 16)
    h3 = _s2conv(h2, layer3_w, layer3_scale, layer3_bias, s // 4, 16, 24)
    h4 = _s2conv(h3, layer4_w, layer4_scale, layer4_bias, s // 8, 24, 32)
    sf, sd = s // 16, s // 4

    wjf = _fold(aspp_proj_w, aspp_proj_scale).reshape(80, 16)
    ha = _aspp(
        h4,
        _fold(aspp0_w, aspp0_scale).reshape(32, 16),
        _fold(aspp1_w, aspp1_scale),
        _fold(aspp2_w[1:2, 1:2], aspp2_scale).reshape(32, 16),
        _fold(aspp3_w[1:2, 1:2], aspp3_scale).reshape(32, 16),
        _fold(aspp_pool_w, aspp_pool_scale).reshape(32, 16),
        [wjf[16 * i:16 * (i + 1), :] for i in range(5)],
        [aspp0_bias, aspp1_bias, aspp2_bias, aspp3_bias, aspp_pool_bias,
         aspp_proj_bias],
        sf, 32, 16)

    hu = _up832(ha, sf, sd, 16)
    lf = _flat1(h2, _fold(dec_low_w, dec_low_scale).reshape(16, 8),
                dec_low_bias, sd)
    d1 = _dec_conv([hu, lf], [16, 8], _fold(dec_conv1_w, dec_conv1_scale),
                   dec_conv1_bias, sd, 16)
    cls = _dec_conv([d1], [16], _fold(dec_conv2_w, dec_conv2_scale),
                    dec_conv2_bias, sd, 16,
                    chain_w=classifier_w.reshape(16, 21),
                    chain_b=classifier_b)
    nc = 21

    rh = jnp.asarray(_interp_mat(s, sd))
    xt = jnp.transpose(cls, (1, 0, 2)).reshape(sd, n * sd * nc)
    t1 = _col_mm(rh, xt, 8192, _BF)
    x2t = jnp.transpose(t1.reshape(s, n, sd, nc), (2, 1, 3, 0)).reshape(sd, -1)
    out = _mmT(x2t, jnp.asarray(_interp_mat(s, sd)).T, 4096, _F32)
    return out.reshape(n, nc, s, s)


# speedup vs baseline: 1.5919x; 1.5919x over previous
import functools

import jax
import jax.numpy as jnp
import numpy as np
from jax.experimental import pallas as pl
from jax.experimental.pallas import tpu as pltpu

_BF = jnp.bfloat16
_F32 = jnp.float32


def _rup(x, m):
    return ((x + m - 1) // m) * m


def _tile(m, target, align=8):
    t = min(target, m)
    t -= t % align
    while t >= align:
        if m % t == 0:
            return t
        t -= align
    return m


def _interp_mat(out_size, in_size):
    if out_size == 1 or in_size == 1:
        m = np.zeros((out_size, in_size), np.float32)
        m[:, 0] = 1.0
        return m
    src = np.arange(out_size, dtype=np.float64) * (in_size - 1) / (out_size - 1)
    i0 = np.clip(np.floor(src).astype(np.int64), 0, in_size - 1)
    i1 = np.clip(i0 + 1, 0, in_size - 1)
    w1 = (src - i0).astype(np.float32)
    w0 = 1.0 - w1
    m = np.zeros((out_size, in_size), np.float32)
    m[np.arange(out_size), i0] += w0
    m[np.arange(out_size), i1] += w1
    return m


def _cparams():
    return pltpu.CompilerParams(
        dimension_semantics=("parallel",),
        vmem_limit_bytes=64 * 1024 * 1024,
    )


def _kron_eye(w2d, blocks):
    k, n = w2d.shape
    mask = np.kron(np.eye(blocks, dtype=np.float32), np.ones((k, n), np.float32))
    return (jnp.asarray(mask)
            * jnp.tile(w2d.astype(_F32), (blocks, blocks))).astype(_BF)


def _fold(w, scale):
    wf = w.astype(_F32)
    if scale is not None:
        wf = wf * scale[None, None, None, :]
    return wf


def _btile(bias, blocks):
    return jnp.tile(bias.astype(_F32), blocks).reshape(1, -1)


def _prep_consts(wp, wo, cin, cout, stride, dil, kp):
    t1 = np.zeros((kp, cin), np.float32)
    t1[:wp * cin] = np.tile(np.eye(cin, dtype=np.float32), (wp, 1))
    t2 = np.tile(np.eye(cout, dtype=np.float32), (1, wo))
    masks = np.zeros((3, kp, wo * cout), np.float32)
    cols = np.arange(wo)
    for kw in range(3):
        msel = np.zeros((wp, wo), np.float32)
        msel[cols * stride + kw * dil, cols] = 1.0
        masks[kw, :wp * cin] = np.kron(msel, np.ones((cin, cout), np.float32))
    return (jnp.asarray(t1).astype(_BF), jnp.asarray(t2).astype(_BF),
            jnp.asarray(masks).astype(_BF))


def _prep_body(*refs, n_items):
    outs = refs[4 * n_items:]
    for idx in range(n_items):
        w9_ref, t1_ref, t2_ref, m_ref = refs[4 * idx:4 * idx + 4]
        for kh in range(3):
            acc = None
            for kw in range(3):
                wt = w9_ref[3 * kh + kw].astype(_BF)
                a = jnp.dot(t1_ref[...], wt, preferred_element_type=_F32)
                b = jnp.dot(a.astype(_BF), t2_ref[...],
                            preferred_element_type=_F32)
                term = b * m_ref[kw].astype(_F32)
                acc = term if acc is None else acc + term
            outs[idx][kh] = acc.astype(_BF)


def _prep_weights(items):
    ops, in_specs, out_shapes, out_specs = [], [], [], []
    for wf, wp, wo, stride, dil, kp in items:
        cin, cout = wf.shape[2], wf.shape[3]
        t1, t2, masks = _prep_consts(wp, wo, cin, cout, stride, dil, kp)
        w9 = wf.reshape(9, cin, cout)
        ops += [w9, t1, t2, masks]
        in_specs += [pl.BlockSpec((9, cin, cout), lambda i: (0, 0, 0)),
                     pl.BlockSpec(t1.shape, lambda i: (0, 0)),
                     pl.BlockSpec(t2.shape, lambda i: (0, 0)),
                     pl.BlockSpec(masks.shape, lambda i: (0, 0, 0))]
        out_shapes.append(jax.ShapeDtypeStruct((3, kp, wo * cout), _BF))
        out_specs.append(pl.BlockSpec((3, kp, wo * cout), lambda i: (0, 0, 0)))
    return pl.pallas_call(
        functools.partial(_prep_body, n_items=len(items)),
        out_shape=tuple(out_shapes),
        grid=(1,),
        in_specs=in_specs,
        out_specs=tuple(out_specs),
        compiler_params=pltpu.CompilerParams(
            dimension_semantics=("arbitrary",),
            vmem_limit_bytes=64 * 1024 * 1024,
        ),
    )(*ops)


def _s2conv_body(x_ref, w_ref, b_ref, o_ref, *, ho, kp):
    nb = o_ref.shape[0]
    xs = x_ref[...]
    acc = None
    for kh in range(3):
        if kh == 0:
            a = xs[:, 0:ho, 0:kp]
        elif kh == 1:
            a = xs[:, 0:ho, kp:2 * kp]
        else:
            a = xs[:, 1:ho + 1, 0:kp]
        d = jnp.dot(a.reshape(nb * ho, kp), w_ref[kh],
                    preferred_element_type=_F32)
        acc = d if acc is None else acc + d
    acc = jnp.maximum(acc + b_ref[...], 0.0)
    o_ref[...] = acc.reshape(nb, ho, acc.shape[-1]).astype(o_ref.dtype)


def _s2conv(x3, wbig, bias, wi, cin, cout):
    n, h, _ = x3.shape
    ho, wo = h // 2, wi // 2
    hp, wp = h + 2, wi + 2
    wpc = wp * cin
    kp = wbig.shape[1]
    xp = jnp.pad(x3, ((0, 0), (1, 1), (cin, kp - wpc + cin)))
    xp = xp.reshape(n, hp // 2, 2 * kp)
    bt = _btile(bias, wo)
    nb = min(max(128 // ho, 1), n)
    while n % nb:
        nb -= 1
    return pl.pallas_call(
        functools.partial(_s2conv_body, ho=ho, kp=kp),
        out_shape=jax.ShapeDtypeStruct((n, ho, wo * cout), _BF),
        grid=(n // nb,),
        in_specs=[pl.BlockSpec((nb, hp // 2, 2 * kp), lambda i: (i, 0, 0)),
                  pl.BlockSpec((3, kp, wo * cout), lambda i: (0, 0, 0)),
                  pl.BlockSpec((1, wo * cout), lambda i: (0, 0))],
        out_specs=pl.BlockSpec((nb, ho, wo * cout), lambda i: (i, 0, 0)),
        compiler_params=_cparams(),
    )(xp, wbig, bt)


def _s1pad(x3, wi, c):
    wpc = (wi + 2) * c
    kp = _rup(wpc, 128)
    return jnp.pad(x3, ((0, 0), (1, 1), (c, kp - wpc + c))), kp


def _dec_body(*refs, n_in, ho, kps, chain):
    x_refs = refs[:n_in]
    w_refs = refs[n_in:2 * n_in]
    b_ref = refs[2 * n_in]
    extra = refs[2 * n_in + 1:]
    nb = extra[-1].shape[0]
    acc = None
    for j in range(n_in):
        xs = x_refs[j][...]
        for kh in range(3):
            a = xs[:, kh:kh + ho, :].reshape(nb * ho, kps[j])
            d = jnp.dot(a, w_refs[j][kh], preferred_element_type=_F32)
            acc = d if acc is None else acc + d
    acc = jnp.maximum(acc + b_ref[...], 0.0)
    if chain:
        wc_ref, bc_ref, o_ref = extra
        acc2 = jnp.dot(acc.astype(_BF), wc_ref[...],
                       preferred_element_type=_F32) + bc_ref[...]
        o_ref[...] = acc2.reshape(nb, ho, acc2.shape[-1]).astype(o_ref.dtype)
    else:
        o_ref = extra[0]
        o_ref[...] = acc.reshape(nb, ho, acc.shape[-1]).astype(o_ref.dtype)


def _dec_conv(x3_list, cins, wbigs, bias, wi, cout, chain_w=None, chain_b=None):
    n, ho = x3_list[0].shape[0], x3_list[0].shape[1]
    xps, kps = [], []
    for x3, cin in zip(x3_list, cins):
        xp, kp = _s1pad(x3, wi, cin)
        xps.append(xp)
        kps.append(kp)
    bt = _btile(bias, wi)
    n_out = wi * cout
    chain = chain_w is not None
    if chain:
        ncls = chain_w.shape[1]
        wc = _kron_eye(chain_w, wi)
        bc = _btile(chain_b, wi)
        n_out = wi * ncls
    nb = min(max(128 // ho, 1), n)
    while n % nb:
        nb -= 1
    in_specs = (
        [pl.BlockSpec((nb, ho + 2, kp), lambda i: (i, 0, 0)) for kp in kps]
        + [pl.BlockSpec((3, kp, wi * cout), lambda i: (0, 0, 0)) for kp in kps]
        + [pl.BlockSpec((1, wi * cout), lambda i: (0, 0))]
    )
    ops = list(xps) + wbigs + [bt]
    if chain:
        in_specs += [pl.BlockSpec((wi * cout, n_out), lambda i: (0, 0)),
                     pl.BlockSpec((1, n_out), lambda i: (0, 0))]
        ops += [wc, bc]
    return pl.pallas_call(
        functools.partial(_dec_body, n_in=len(x3_list), ho=ho,
                          kps=tuple(kps), chain=chain),
        out_shape=jax.ShapeDtypeStruct((n, ho, n_out), _BF),
        grid=(n // nb,),
        in_specs=in_specs,
        out_specs=pl.BlockSpec((nb, ho, n_out), lambda i: (i, 0, 0)),
        compiler_params=_cparams(),
    )(*ops)


def _flat1_body(x_ref, w_ref, b_ref, o_ref):
    nb, ho, kp = x_ref.shape
    a = x_ref[...].reshape(nb * ho, kp)
    acc = jnp.maximum(jnp.dot(a, w_ref[...], preferred_element_type=_F32)
                      + b_ref[...], 0.0)
    o_ref[...] = acc.reshape(nb, ho, acc.shape[-1]).astype(o_ref.dtype)


def _flat1(x3, w2d, bias, wi):
    n, ho, _ = x3.shape
    wk = _kron_eye(w2d, wi)
    bt = _btile(bias, wi)
    n_out = wk.shape[1]
    nb = min(max(256 // ho, 1), n)
    while n % nb:
        nb -= 1
    return pl.pallas_call(
        _flat1_body,
        out_shape=jax.ShapeDtypeStruct((n, ho, n_out), _BF),
        grid=(n // nb,),
        in_specs=[pl.BlockSpec((nb, ho, x3.shape[2]), lambda i: (i, 0, 0)),
                  pl.BlockSpec((wk.shape[0], n_out), lambda i: (0, 0)),
                  pl.BlockSpec((1, n_out), lambda i: (0, 0))],
        out_specs=pl.BlockSpec((nb, ho, n_out), lambda i: (i, 0, 0)),
        compiler_params=_cparams(),
    )(x3, wk, bt)


def _aspp_body(h_ref, hp6_ref, w0_ref, w2_ref, w3_ref, wb1_ref, wp_ref,
               j0_ref, j1_ref, j2_ref, j3_ref, j4_ref, k8_ref,
               p2_ref, c8_ref, e2_ref,
               c0_ref, c1_ref, c2_ref, c3_ref, cp_ref, cj_ref, o_ref):
    nb, sf, lanes = o_ref.shape
    h = h_ref[...]
    b0 = jnp.maximum(jnp.dot(h, w0_ref[...], preferred_element_type=_F32)
                     + c0_ref[...], 0.0).astype(_BF)
    b2 = jnp.maximum(jnp.dot(h, w2_ref[...], preferred_element_type=_F32)
                     + c2_ref[...], 0.0).astype(_BF)
    b3 = jnp.maximum(jnp.dot(h, w3_ref[...], preferred_element_type=_F32)
                     + c3_ref[...], 0.0).astype(_BF)
    hp = hp6_ref[...]
    b1 = None
    for kh in range(3):
        a = hp[:, 6 * kh:6 * kh + sf, :].reshape(nb * sf, hp.shape[-1])
        d = jnp.dot(a, wb1_ref[kh], preferred_element_type=_F32)
        b1 = d if b1 is None else b1 + d
    b1 = jnp.maximum(b1 + c1_ref[...], 0.0).astype(_BF)
    acc = jnp.dot(b0, j0_ref[...], preferred_element_type=_F32)
    acc = acc + jnp.dot(b1, j1_ref[...], preferred_element_type=_F32)
    acc = acc + jnp.dot(b2, j2_ref[...], preferred_element_type=_F32)
    acc = acc + jnp.dot(b3, j3_ref[...], preferred_element_type=_F32)
    pr = jnp.dot(p2_ref[...], h, preferred_element_type=_F32)
    pm = jnp.dot(pr.astype(_BF), c8_ref[...], preferred_element_type=_F32)
    b4 = jnp.maximum(jnp.dot(pm.astype(_BF), wp_ref[...],
                             preferred_element_type=_F32) + cp_ref[...], 0.0)
    c4 = jnp.dot(b4.astype(_BF), j4_ref[...], preferred_element_type=_F32)
    c4t = jnp.dot(c4.astype(_BF), k8_ref[...], preferred_element_type=_F32)
    acc = acc + jnp.dot(e2_ref[...], c4t.astype(_BF),
                        preferred_element_type=_F32)
    acc = jnp.maximum(acc + cj_ref[...], 0.0)
    o_ref[...] = acc.reshape(nb, sf, lanes).astype(o_ref.dtype)


def _aspp(h4, w0, wb1, w2, w3, wp, wj, biases, sf, cm, co):
    n = h4.shape[0]
    hflat = h4.reshape(n * sf, sf * cm)
    hp6 = jnp.pad(h4, ((0, 0), (6, 6), (6 * cm, 6 * cm)))
    g = 2 if n % 2 == 0 else 1
    nb = n // g
    k8 = np.zeros((co * sf, co * sf), np.float32)
    for wi_ in range(sf):
        k8[0:co, wi_ * co:(wi_ + 1) * co] = np.eye(co)
    p2 = np.kron(np.eye(n, dtype=np.float32), np.full((1, sf), 1.0 / sf))
    c8 = np.kron(np.full((sf, 1), 1.0 / sf, np.float32), np.eye(cm))
    e2 = np.kron(np.eye(n, dtype=np.float32), np.ones((sf, 1), np.float32))
    c0, c1, c2, c3 = [_btile(b, sf) for b in biases[:4]]
    cp = jnp.pad(biases[4].astype(_F32).reshape(1, -1),
                 ((0, 0), (0, co * sf - co)))
    cj = _btile(biases[5], sf)
    jpads = [_kron_eye(w, sf) for w in wj[:4]] + [
        jnp.pad(wj[4].astype(_F32), ((0, co * sf - wj[4].shape[0]),
                                     (0, co * sf - wj[4].shape[1]))).astype(_BF)]
    wpp = jnp.pad(wp.astype(_F32), ((0, 0), (0, co * sf - co))).astype(_BF)
    lanes = sf * co
    in_specs = [
        pl.BlockSpec((nb * sf, sf * cm), lambda i: (i, 0)),
        pl.BlockSpec((nb, sf + 12, hp6.shape[2]), lambda i: (i, 0, 0)),
        pl.BlockSpec((sf * cm, lanes), lambda i: (0, 0)),
        pl.BlockSpec((sf * cm, lanes), lambda i: (0, 0)),
        pl.BlockSpec((sf * cm, lanes), lambda i: (0, 0)),
        pl.BlockSpec((3, hp6.shape[2], lanes), lambda i: (0, 0, 0)),
        pl.BlockSpec((cm, lanes), lambda i: (0, 0)),
    ] + [pl.BlockSpec((lanes, lanes), lambda i: (0, 0))] * 6 + [
        pl.BlockSpec((n, nb * sf), lambda i: (0, i)),
        pl.BlockSpec((sf * cm, cm), lambda i: (0, 0)),
        pl.BlockSpec((nb * sf, n), lambda i: (i, 0)),
    ] + [pl.BlockSpec((1, lanes), lambda i: (0, 0))] * 6
    return pl.pallas_call(
        _aspp_body,
        out_shape=jax.ShapeDtypeStruct((n, sf, lanes), _BF),
        grid=(g,),
        in_specs=in_specs,
        out_specs=pl.BlockSpec((nb, sf, lanes), lambda i: (i, 0, 0)),
        compiler_params=_cparams(),
    )(hflat, hp6,
      _kron_eye(w0, sf), _kron_eye(w2, sf), _kron_eye(w3, sf), wb1, wpp,
      *jpads, jnp.asarray(k8).astype(_BF),
      jnp.asarray(p2).astype(_BF), jnp.asarray(c8).astype(_BF),
      jnp.asarray(e2).astype(_BF),
      c0, c1, c2, c3, cp, cj)


def _up_body(x_ref, ww_ref, rh_ref, o_ref):
    nb, ho, lanes = o_ref.shape
    sf = x_ref.shape[1]
    xm = jnp.dot(x_ref[...].reshape(nb * sf, x_ref.shape[2]), ww_ref[...],
                 preferred_element_type=_F32)
    hu = jnp.dot(rh_ref[...], xm.astype(_BF), preferred_element_type=_F32)
    o_ref[...] = hu.reshape(nb, ho, lanes).astype(o_ref.dtype)


def _up832(x3, sf, sd, co):
    n = x3.shape[0]
    r1 = _interp_mat(sd, sf)
    ww = np.einsum('ow,ij->wioj', r1, np.eye(co, dtype=np.float32))
    ww = jnp.asarray(ww.reshape(sf * co, sd * co)).astype(_BF)
    bigrh = jnp.asarray(np.kron(np.eye(n, dtype=np.float32), r1)).astype(_BF)
    g = 2 if n % 2 == 0 else 1
    nb = n // g
    return pl.pallas_call(
        _up_body,
        out_shape=jax.ShapeDtypeStruct((n, sd, sd * co), _BF),
        grid=(g,),
        in_specs=[pl.BlockSpec((nb, sf, sf * co), lambda i: (i, 0, 0)),
                  pl.BlockSpec((sf * co, sd * co), lambda i: (0, 0)),
                  pl.BlockSpec((nb * sd, nb * sf), lambda i: (i, i))],
        out_specs=pl.BlockSpec((nb, sd, sd * co), lambda i: (i, 0, 0)),
        compiler_params=_cparams(),
    )(x3, ww, bigrh)


def _mm_body(a_ref, b_ref, o_ref):
    o_ref[...] = jnp.dot(a_ref[...], b_ref[...],
                         preferred_element_type=_F32).astype(o_ref.dtype)


def _mmT_body(a_ref, b_ref, o_ref):
    o_ref[...] = jax.lax.dot_general(
        a_ref[...], b_ref[...], (((0,), (0,)), ((), ())),
        preferred_element_type=_F32).astype(o_ref.dtype)


def _mmT(at, b, tile_m, out_dtype):
    k, m = at.shape
    n = b.shape[1]
    tm = _tile(m, tile_m, align=128)
    return pl.pallas_call(
        _mmT_body,
        out_shape=jax.ShapeDtypeStruct((m, n), out_dtype),
        grid=(m // tm,),
        in_specs=[pl.BlockSpec((k, tm), lambda i: (0, i)),
                  pl.BlockSpec((k, n), lambda i: (0, 0))],
        out_specs=pl.BlockSpec((tm, n), lambda i: (i, 0)),
        compiler_params=_cparams(),
    )(at.astype(_BF), b.astype(_BF))


def _col_mm(a, b, tile_n, out_dtype):
    m, k = a.shape
    n = b.shape[1]
    tn = _tile(n, tile_n, align=128)
    return pl.pallas_call(
        _mm_body,
        out_shape=jax.ShapeDtypeStruct((m, n), out_dtype),
        grid=(n // tn,),
        in_specs=[pl.BlockSpec((m, k), lambda j: (0, 0)),
                  pl.BlockSpec((k, tn), lambda j: (0, j))],
        out_specs=pl.BlockSpec((m, tn), lambda j: (0, j)),
        compiler_params=_cparams(),
    )(a.astype(_BF), b.astype(_BF))


def kernel(stem1_w, stem1_scale, stem1_bias, stem2_w, stem2_scale, stem2_bias,
           layer3_w, layer3_scale, layer3_bias, layer4_w, layer4_scale,
           layer4_bias, aspp0_w, aspp0_scale, aspp0_bias, aspp1_w, aspp1_scale,
           aspp1_bias, aspp2_w, aspp2_scale, aspp2_bias, aspp3_w, aspp3_scale,
           aspp3_bias, aspp_pool_w, aspp_pool_scale, aspp_pool_bias,
           aspp_proj_w, aspp_proj_scale, aspp_proj_bias, dec_low_w,
           dec_low_scale, dec_low_bias, dec_conv1_w, dec_conv1_scale,
           dec_conv1_bias, dec_conv2_w, dec_conv2_scale, dec_conv2_bias,
           classifier_w, classifier_b, x):
    n, _, s, _ = x.shape
    xh = jnp.transpose(x, (0, 2, 3, 1)).astype(_BF).reshape(n, s, s * 3)
    sf, sd = s // 16, s // 4

    wf1 = _fold(dec_conv1_w, dec_conv1_scale)
    wbigs = _prep_weights([
        (_fold(stem1_w, stem1_scale), s + 2, s // 2, 2, 1,
         _rup((s + 2) * 3, 128)),
        (_fold(stem2_w, stem2_scale), s // 2 + 2, s // 4, 2, 1,
         _rup((s // 2 + 2) * 8, 128)),
        (_fold(layer3_w, layer3_scale), s // 4 + 2, s // 8, 2, 1,
         _rup((s // 4 + 2) * 16, 128)),
        (_fold(layer4_w, layer4_scale), s // 8 + 2, sf, 2, 1,
         _rup((s // 8 + 2) * 24, 128)),
        (wf1[:, :, :16, :], sd + 2, sd, 1, 1, _rup((sd + 2) * 16, 128)),
        (wf1[:, :, 16:, :], sd + 2, sd, 1, 1, _rup((sd + 2) * 8, 128)),
        (_fold(dec_conv2_w, dec_conv2_scale), sd + 2, sd, 1, 1,
         _rup((sd + 2) * 16, 128)),
        (_fold(aspp1_w, aspp1_scale), sf + 12, sf, 1, 6, (sf + 12) * 32),
    ])

    h1 = _s2conv(xh, wbigs[0], stem1_bias, s, 3, 8)
    h2 = _s2conv(h1, wbigs[1], stem2_bias, s // 2, 8, 16)
    h3 = _s2conv(h2, wbigs[2], layer3_bias, s // 4, 16, 24)
    h4 = _s2conv(h3, wbigs[3], layer4_bias, s // 8, 24, 32)

    wjf = _fold(aspp_proj_w, aspp_proj_scale).reshape(80, 16)
    ha = _aspp(
        h4,
        _fold(aspp0_w, aspp0_scale).reshape(32, 16),
        wbigs[7],
        _fold(aspp2_w[1:2, 1:2], aspp2_scale).reshape(32, 16),
        _fold(aspp3_w[1:2, 1:2], aspp3_scale).reshape(32, 16),
        _fold(aspp_pool_w, aspp_pool_scale).reshape(32, 16),
        [wjf[16 * i:16 * (i + 1), :] for i in range(5)],
        [aspp0_bias, aspp1_bias, aspp2_bias, aspp3_bias, aspp_pool_bias,
         aspp_proj_bias],
        sf, 32, 16)

    hu = _up832(ha, sf, sd, 16)
    lf = _flat1(h2, _fold(dec_low_w, dec_low_scale).reshape(16, 8),
                dec_low_bias, sd)
    d1 = _dec_conv([hu, lf], [16, 8], [wbigs[4], wbigs[5]],
                   dec_conv1_bias, sd, 16)
    cls = _dec_conv([d1], [16], [wbigs[6]],
                    dec_conv2_bias, sd, 16,
                    chain_w=classifier_w.reshape(16, 21),
                    chain_b=classifier_b)
    nc = 21

    rh = jnp.asarray(_interp_mat(s, sd))
    xt = jnp.transpose(cls, (1, 0, 2)).reshape(sd, n * sd * nc)
    t1 = _col_mm(rh, xt, 8192, _BF)
    x2t = jnp.transpose(t1.reshape(s, n, sd, nc), (2, 1, 3, 0)).reshape(sd, -1)
    out = _mmT(x2t, jnp.asarray(_interp_mat(s, sd)).T, 4096, _F32)
    return out.reshape(n, nc, s, s)


# speedup vs baseline: 2.1424x; 1.3458x over previous
import functools

import jax
import jax.numpy as jnp
import numpy as np
from jax.experimental import pallas as pl
from jax.experimental.pallas import tpu as pltpu

_BF = jnp.bfloat16
_F32 = jnp.float32


def _rup(x, m):
    return ((x + m - 1) // m) * m


def _tile(m, target, align=8):
    t = min(target, m)
    t -= t % align
    while t >= align:
        if m % t == 0:
            return t
        t -= align
    return m


def _interp_mat(out_size, in_size):
    if out_size == 1 or in_size == 1:
        m = np.zeros((out_size, in_size), np.float32)
        m[:, 0] = 1.0
        return m
    src = np.arange(out_size, dtype=np.float64) * (in_size - 1) / (out_size - 1)
    i0 = np.clip(np.floor(src).astype(np.int64), 0, in_size - 1)
    i1 = np.clip(i0 + 1, 0, in_size - 1)
    w1 = (src - i0).astype(np.float32)
    w0 = 1.0 - w1
    m = np.zeros((out_size, in_size), np.float32)
    m[np.arange(out_size), i0] += w0
    m[np.arange(out_size), i1] += w1
    return m


def _cparams():
    return pltpu.CompilerParams(
        dimension_semantics=("parallel",),
        vmem_limit_bytes=64 * 1024 * 1024,
    )


def _kron_eye(w2d, blocks):
    k, n = w2d.shape
    mask = np.kron(np.eye(blocks, dtype=np.float32), np.ones((k, n), np.float32))
    return (jnp.asarray(mask)
            * jnp.tile(w2d.astype(_F32), (blocks, blocks))).astype(_BF)


def _fold(w, scale):
    wf = w.astype(_F32)
    if scale is not None:
        wf = wf * scale[None, None, None, :]
    return wf


def _btile(bias, blocks):
    return jnp.tile(bias.astype(_F32), blocks).reshape(1, -1)


def _prep_consts(wp, wo, cin, cout, stride, dil, kp):
    t1 = np.zeros((kp, cin), np.float32)
    t1[:wp * cin] = np.tile(np.eye(cin, dtype=np.float32), (wp, 1))
    t2 = np.tile(np.eye(cout, dtype=np.float32), (1, wo))
    masks = np.zeros((3, kp, wo * cout), np.float32)
    cols = np.arange(wo)
    for kw in range(3):
        msel = np.zeros((wp, wo), np.float32)
        msel[cols * stride + kw * dil, cols] = 1.0
        masks[kw, :wp * cin] = np.kron(msel, np.ones((cin, cout), np.float32))
    return (jnp.asarray(t1).astype(_BF), jnp.asarray(t2).astype(_BF),
            jnp.asarray(masks).astype(_BF))


def _prep_body(*refs, shapes):
    outs = refs[4 * len(shapes):]
    for idx, (ot, q) in enumerate(shapes):
        w_ref, t1_ref, t2_ref, m_ref = refs[4 * idx:4 * idx + 4]
        for kh in range(ot):
            acc = None
            for kw in range(q):
                wt = w_ref[q * kh + kw].astype(_BF)
                a = jnp.dot(t1_ref[...], wt, preferred_element_type=_F32)
                b = jnp.dot(a.astype(_BF), t2_ref[...],
                            preferred_element_type=_F32)
                term = b * m_ref[kw].astype(_F32)
                acc = term if acc is None else acc + term
            outs[idx][kh] = acc.astype(_BF)


def _prep_weights(items):
    ops, in_specs, out_shapes, out_specs, shapes = [], [], [], [], []
    for w_flat, t1, t2, masks, ot in items:
        ops += [w_flat, t1, t2, masks]
        in_specs += [pl.BlockSpec(w_flat.shape, lambda i: (0, 0, 0)),
                     pl.BlockSpec(t1.shape, lambda i: (0, 0)),
                     pl.BlockSpec(t2.shape, lambda i: (0, 0)),
                     pl.BlockSpec(masks.shape, lambda i: (0, 0, 0))]
        out_shapes.append(jax.ShapeDtypeStruct(
            (ot, masks.shape[1], masks.shape[2]), _BF))
        out_specs.append(pl.BlockSpec(
            (ot, masks.shape[1], masks.shape[2]), lambda i: (0, 0, 0)))
        shapes.append((ot, masks.shape[0]))
    return pl.pallas_call(
        functools.partial(_prep_body, shapes=tuple(shapes)),
        out_shape=tuple(out_shapes),
        grid=(1,),
        in_specs=in_specs,
        out_specs=tuple(out_specs),
        compiler_params=pltpu.CompilerParams(
            dimension_semantics=("arbitrary",),
            vmem_limit_bytes=64 * 1024 * 1024,
        ),
    )(*ops)


def _conv_item(wf, wp, wo, stride, dil, kp):
    cin, cout = wf.shape[2], wf.shape[3]
    t1, t2, masks = _prep_consts(wp, wo, cin, cout, stride, dil, kp)
    return (wf.reshape(9, cin, cout), t1, t2, masks, 3)


def _cls_item(wcls, wi):
    ci, nc = wcls.shape
    t1 = jnp.asarray(np.tile(np.eye(ci, dtype=np.float32), (wi, 1))).astype(_BF)
    t2 = jnp.asarray(np.kron(np.eye(nc, dtype=np.float32),
                             np.ones((1, wi), np.float32))).astype(_BF)
    mask = np.zeros((wi, ci, nc, wi), np.float32)
    for w in range(wi):
        mask[w, :, :, w] = 1.0
    masks = jnp.asarray(mask.reshape(1, wi * ci, nc * wi)).astype(_BF)
    return (wcls.astype(_F32).reshape(1, ci, nc), t1, t2, masks, 1)


def _s2conv_body(x_ref, w_ref, b_ref, o_ref, *, ho, kp):
    nb = o_ref.shape[0]
    xs = x_ref[...]
    acc = None
    for kh in range(3):
        if kh == 0:
            a = xs[:, 0:ho, 0:kp]
        elif kh == 1:
            a = xs[:, 0:ho, kp:2 * kp]
        else:
            a = xs[:, 1:ho + 1, 0:kp]
        d = jnp.dot(a.reshape(nb * ho, kp), w_ref[kh],
                    preferred_element_type=_F32)
        acc = d if acc is None else acc + d
    acc = jnp.maximum(acc + b_ref[...], 0.0)
    o_ref[...] = acc.reshape(nb, ho, acc.shape[-1]).astype(o_ref.dtype)


def _s2conv(x3, wbig, bias, wi, cin, cout):
    n, h, _ = x3.shape
    ho, wo = h // 2, wi // 2
    hp, wp = h + 2, wi + 2
    wpc = wp * cin
    kp = wbig.shape[1]
    xp = jnp.pad(x3, ((0, 0), (1, 1), (cin, kp - wpc + cin)))
    xp = xp.reshape(n, hp // 2, 2 * kp)
    bt = _btile(bias, wo)
    nb = min(max(128 // ho, 1), n)
    while n % nb:
        nb -= 1
    return pl.pallas_call(
        functools.partial(_s2conv_body, ho=ho, kp=kp),
        out_shape=jax.ShapeDtypeStruct((n, ho, wo * cout), _BF),
        grid=(n // nb,),
        in_specs=[pl.BlockSpec((nb, hp // 2, 2 * kp), lambda i: (i, 0, 0)),
                  pl.BlockSpec((3, kp, wo * cout), lambda i: (0, 0, 0)),
                  pl.BlockSpec((1, wo * cout), lambda i: (0, 0))],
        out_specs=pl.BlockSpec((nb, ho, wo * cout), lambda i: (i, 0, 0)),
        compiler_params=_cparams(),
    )(xp, wbig, bt)


def _s1pad(x3, wi, c):
    wpc = (wi + 2) * c
    kp = _rup(wpc, 128)
    return jnp.pad(x3, ((0, 0), (1, 1), (c, kp - wpc + c))), kp


def _dec_body(*refs, n_in, ho, kps, chain):
    x_refs = refs[:n_in]
    w_refs = refs[n_in:2 * n_in]
    b_ref = refs[2 * n_in]
    extra = refs[2 * n_in + 1:]
    nb = extra[-1].shape[0]
    acc = None
    for j in range(n_in):
        xs = x_refs[j][...]
        for kh in range(3):
            a = xs[:, kh:kh + ho, :].reshape(nb * ho, kps[j])
            d = jnp.dot(a, w_refs[j][kh], preferred_element_type=_F32)
            acc = d if acc is None else acc + d
    acc = jnp.maximum(acc + b_ref[...], 0.0)
    if chain:
        wc_ref, bc_ref, o_ref = extra
        acc2 = jnp.dot(acc.astype(_BF), wc_ref[...],
                       preferred_element_type=_F32) + bc_ref[...]
        o_ref[...] = acc2.reshape(nb, ho, acc2.shape[-1]).astype(o_ref.dtype)
    else:
        o_ref = extra[0]
        o_ref[...] = acc.reshape(nb, ho, acc.shape[-1]).astype(o_ref.dtype)


def _dec_conv(x3_list, cins, wbigs, bias, wi, cout, chain_w=None, chain_b=None):
    n, ho = x3_list[0].shape[0], x3_list[0].shape[1]
    xps, kps = [], []
    for x3, cin in zip(x3_list, cins):
        xp, kp = _s1pad(x3, wi, cin)
        xps.append(xp)
        kps.append(kp)
    bt = _btile(bias, wi)
    n_out = wi * cout
    chain = chain_w is not None
    if chain:
        wc, bc = chain_w, chain_b
        n_out = wc.shape[1]
    nb = min(max(128 // ho, 1), n)
    while n % nb:
        nb -= 1
    in_specs = (
        [pl.BlockSpec((nb, ho + 2, kp), lambda i: (i, 0, 0)) for kp in kps]
        + [pl.BlockSpec((3, kp, wi * cout), lambda i: (0, 0, 0)) for kp in kps]
        + [pl.BlockSpec((1, wi * cout), lambda i: (0, 0))]
    )
    ops = list(xps) + wbigs + [bt]
    if chain:
        in_specs += [pl.BlockSpec((wi * cout, n_out), lambda i: (0, 0)),
                     pl.BlockSpec((1, n_out), lambda i: (0, 0))]
        ops += [wc, bc]
    return pl.pallas_call(
        functools.partial(_dec_body, n_in=len(x3_list), ho=ho,
                          kps=tuple(kps), chain=chain),
        out_shape=jax.ShapeDtypeStruct((n, ho, n_out), _BF),
        grid=(n // nb,),
        in_specs=in_specs,
        out_specs=pl.BlockSpec((nb, ho, n_out), lambda i: (i, 0, 0)),
        compiler_params=_cparams(),
    )(*ops)


def _flat1_body(x_ref, w_ref, b_ref, o_ref):
    nb, ho, kp = x_ref.shape
    a = x_ref[...].reshape(nb * ho, kp)
    acc = jnp.maximum(jnp.dot(a, w_ref[...], preferred_element_type=_F32)
                      + b_ref[...], 0.0)
    o_ref[...] = acc.reshape(nb, ho, acc.shape[-1]).astype(o_ref.dtype)


def _flat1(x3, w2d, bias, wi):
    n, ho, _ = x3.shape
    wk = _kron_eye(w2d, wi)
    bt = _btile(bias, wi)
    n_out = wk.shape[1]
    nb = min(max(256 // ho, 1), n)
    while n % nb:
        nb -= 1
    return pl.pallas_call(
        _flat1_body,
        out_shape=jax.ShapeDtypeStruct((n, ho, n_out), _BF),
        grid=(n // nb,),
        in_specs=[pl.BlockSpec((nb, ho, x3.shape[2]), lambda i: (i, 0, 0)),
                  pl.BlockSpec((wk.shape[0], n_out), lambda i: (0, 0)),
                  pl.BlockSpec((1, n_out), lambda i: (0, 0))],
        out_specs=pl.BlockSpec((nb, ho, n_out), lambda i: (i, 0, 0)),
        compiler_params=_cparams(),
    )(x3, wk, bt)


def _aspp_body(h_ref, hp6_ref, w0_ref, w2_ref, w3_ref, wb1_ref, wp_ref,
               j0_ref, j1_ref, j2_ref, j3_ref, j4_ref, k8_ref,
               p2_ref, c8_ref, e2_ref,
               c0_ref, c1_ref, c2_ref, c3_ref, cp_ref, cj_ref, o_ref):
    nb, sf, lanes = o_ref.shape
    h = h_ref[...]
    b0 = jnp.maximum(jnp.dot(h, w0_ref[...], preferred_element_type=_F32)
                     + c0_ref[...], 0.0).astype(_BF)
    b2 = jnp.maximum(jnp.dot(h, w2_ref[...], preferred_element_type=_F32)
                     + c2_ref[...], 0.0).astype(_BF)
    b3 = jnp.maximum(jnp.dot(h, w3_ref[...], preferred_element_type=_F32)
                     + c3_ref[...], 0.0).astype(_BF)
    hp = hp6_ref[...]
    b1 = None
    for kh in range(3):
        a = hp[:, 6 * kh:6 * kh + sf, :].reshape(nb * sf, hp.shape[-1])
        d = jnp.dot(a, wb1_ref[kh], preferred_element_type=_F32)
        b1 = d if b1 is None else b1 + d
    b1 = jnp.maximum(b1 + c1_ref[...], 0.0).astype(_BF)
    acc = jnp.dot(b0, j0_ref[...], preferred_element_type=_F32)
    acc = acc + jnp.dot(b1, j1_ref[...], preferred_element_type=_F32)
    acc = acc + jnp.dot(b2, j2_ref[...], preferred_element_type=_F32)
    acc = acc + jnp.dot(b3, j3_ref[...], preferred_element_type=_F32)
    pr = jnp.dot(p2_ref[...], h, preferred_element_type=_F32)
    pm = jnp.dot(pr.astype(_BF), c8_ref[...], preferred_element_type=_F32)
    b4 = jnp.maximum(jnp.dot(pm.astype(_BF), wp_ref[...],
                             preferred_element_type=_F32) + cp_ref[...], 0.0)
    c4 = jnp.dot(b4.astype(_BF), j4_ref[...], preferred_element_type=_F32)
    c4t = jnp.dot(c4.astype(_BF), k8_ref[...], preferred_element_type=_F32)
    acc = acc + jnp.dot(e2_ref[...], c4t.astype(_BF),
                        preferred_element_type=_F32)
    acc = jnp.maximum(acc + cj_ref[...], 0.0)
    o_ref[...] = acc.reshape(nb, sf, lanes).astype(o_ref.dtype)


def _aspp(h4, w0, wb1, w2, w3, wp, wj, biases, sf, cm, co):
    n = h4.shape[0]
    hflat = h4.reshape(n * sf, sf * cm)
    hp6 = jnp.pad(h4, ((0, 0), (6, 6), (6 * cm, 6 * cm)))
    g = 2 if n % 2 == 0 else 1
    nb = n // g
    k8 = np.zeros((co * sf, co * sf), np.float32)
    for wi_ in range(sf):
        k8[0:co, wi_ * co:(wi_ + 1) * co] = np.eye(co)
    p2 = np.kron(np.eye(n, dtype=np.float32), np.full((1, sf), 1.0 / sf))
    c8 = np.kron(np.full((sf, 1), 1.0 / sf, np.float32), np.eye(cm))
    e2 = np.kron(np.eye(n, dtype=np.float32), np.ones((sf, 1), np.float32))
    c0, c1, c2, c3 = [_btile(b, sf) for b in biases[:4]]
    cp = jnp.pad(biases[4].astype(_F32).reshape(1, -1),
                 ((0, 0), (0, co * sf - co)))
    cj = _btile(biases[5], sf)
    jpads = [_kron_eye(w, sf) for w in wj[:4]] + [
        jnp.pad(wj[4].astype(_F32), ((0, co * sf - wj[4].shape[0]),
                                     (0, co * sf - wj[4].shape[1]))).astype(_BF)]
    wpp = jnp.pad(wp.astype(_F32), ((0, 0), (0, co * sf - co))).astype(_BF)
    lanes = sf * co
    in_specs = [
        pl.BlockSpec((nb * sf, sf * cm), lambda i: (i, 0)),
        pl.BlockSpec((nb, sf + 12, hp6.shape[2]), lambda i: (i, 0, 0)),
        pl.BlockSpec((sf * cm, lanes), lambda i: (0, 0)),
        pl.BlockSpec((sf * cm, lanes), lambda i: (0, 0)),
        pl.BlockSpec((sf * cm, lanes), lambda i: (0, 0)),
        pl.BlockSpec((3, hp6.shape[2], lanes), lambda i: (0, 0, 0)),
        pl.BlockSpec((cm, lanes), lambda i: (0, 0)),
    ] + [pl.BlockSpec((lanes, lanes), lambda i: (0, 0))] * 6 + [
        pl.BlockSpec((n, nb * sf), lambda i: (0, i)),
        pl.BlockSpec((sf * cm, cm), lambda i: (0, 0)),
        pl.BlockSpec((nb * sf, n), lambda i: (i, 0)),
    ] + [pl.BlockSpec((1, lanes), lambda i: (0, 0))] * 6
    return pl.pallas_call(
        _aspp_body,
        out_shape=jax.ShapeDtypeStruct((n, sf, lanes), _BF),
        grid=(g,),
        in_specs=in_specs,
        out_specs=pl.BlockSpec((nb, sf, lanes), lambda i: (i, 0, 0)),
        compiler_params=_cparams(),
    )(hflat, hp6,
      _kron_eye(w0, sf), _kron_eye(w2, sf), _kron_eye(w3, sf), wb1, wpp,
      *jpads, jnp.asarray(k8).astype(_BF),
      jnp.asarray(p2).astype(_BF), jnp.asarray(c8).astype(_BF),
      jnp.asarray(e2).astype(_BF),
      c0, c1, c2, c3, cp, cj)


def _up_body(x_ref, ww_ref, rh_ref, o_ref):
    nb, ho, lanes = o_ref.shape
    sf = x_ref.shape[1]
    xm = jnp.dot(x_ref[...].reshape(nb * sf, x_ref.shape[2]), ww_ref[...],
                 preferred_element_type=_F32)
    hu = jnp.dot(rh_ref[...], xm.astype(_BF), preferred_element_type=_F32)
    o_ref[...] = hu.reshape(nb, ho, lanes).astype(o_ref.dtype)


def _up832(x3, sf, sd, co):
    n = x3.shape[0]
    r1 = _interp_mat(sd, sf)
    ww = np.einsum('ow,ij->wioj', r1, np.eye(co, dtype=np.float32))
    ww = jnp.asarray(ww.reshape(sf * co, sd * co)).astype(_BF)
    bigrh = jnp.asarray(np.kron(np.eye(n, dtype=np.float32), r1)).astype(_BF)
    g = 2 if n % 2 == 0 else 1
    nb = n // g
    return pl.pallas_call(
        _up_body,
        out_shape=jax.ShapeDtypeStruct((n, sd, sd * co), _BF),
        grid=(g,),
        in_specs=[pl.BlockSpec((nb, sf, sf * co), lambda i: (i, 0, 0)),
                  pl.BlockSpec((sf * co, sd * co), lambda i: (0, 0)),
                  pl.BlockSpec((nb * sd, nb * sf), lambda i: (i, i))],
        out_specs=pl.BlockSpec((nb, sd, sd * co), lambda i: (i, 0, 0)),
        compiler_params=_cparams(),
    )(x3, ww, bigrh)


def _finup_body(c_ref, rhk_ref, rwt_ref, o_ref, *, nc, sd):
    nb, _, lanes = c_ref.shape
    s = o_ref.shape[2]
    cm = c_ref[...].reshape(nb * sd, lanes)
    t1 = jax.lax.dot_general(cm, rhk_ref[...], (((0,), (0,)), ((), ())),
                             preferred_element_type=_F32)
    t1 = t1.astype(_BF)
    for c in range(nc):
        tc = t1[sd * c:sd * (c + 1), :]
        oc = jax.lax.dot_general(tc, rwt_ref[...], (((0,), (0,)), ((), ())),
                                 preferred_element_type=_F32)
        o_ref[:, c, :, :] = oc.reshape(nb, s, s)


def _finup(cls3, n, s, sd, nc):
    nb = 2
    while n % nb:
        nb -= 1
    rh = _interp_mat(s, sd)
    rhk = jnp.asarray(np.kron(np.eye(nb, dtype=np.float32), rh.T)).astype(_BF)
    rwt = jnp.asarray(rh.T).astype(_BF)
    return pl.pallas_call(
        functools.partial(_finup_body, nc=nc, sd=sd),
        out_shape=jax.ShapeDtypeStruct((n, nc, s, s), _F32),
        grid=(n // nb,),
        in_specs=[pl.BlockSpec((nb, sd, nc * sd), lambda i: (i, 0, 0)),
                  pl.BlockSpec((nb * sd, nb * s), lambda i: (0, 0)),
                  pl.BlockSpec((sd, s), lambda i: (0, 0))],
        out_specs=pl.BlockSpec((nb, nc, s, s), lambda i: (i, 0, 0, 0)),
        compiler_params=_cparams(),
    )(cls3, rhk, rwt)


def _mm_body(a_ref, b_ref, o_ref):
    o_ref[...] = jnp.dot(a_ref[...], b_ref[...],
                         preferred_element_type=_F32).astype(o_ref.dtype)


def _mmT_body(a_ref, b_ref, o_ref):
    o_ref[...] = jax.lax.dot_general(
        a_ref[...], b_ref[...], (((0,), (0,)), ((), ())),
        preferred_element_type=_F32).astype(o_ref.dtype)


def _mmT(at, b, tile_m, out_dtype):
    k, m = at.shape
    n = b.shape[1]
    tm = _tile(m, tile_m, align=128)
    return pl.pallas_call(
        _mmT_body,
        out_shape=jax.ShapeDtypeStruct((m, n), out_dtype),
        grid=(m // tm,),
        in_specs=[pl.BlockSpec((k, tm), lambda i: (0, i)),
                  pl.BlockSpec((k, n), lambda i: (0, 0))],
        out_specs=pl.BlockSpec((tm, n), lambda i: (i, 0)),
        compiler_params=_cparams(),
    )(at.astype(_BF), b.astype(_BF))


def _col_mm(a, b, tile_n, out_dtype):
    m, k = a.shape
    n = b.shape[1]
    tn = _tile(n, tile_n, align=128)
    return pl.pallas_call(
        _mm_body,
        out_shape=jax.ShapeDtypeStruct((m, n), out_dtype),
        grid=(n // tn,),
        in_specs=[pl.BlockSpec((m, k), lambda j: (0, 0)),
                  pl.BlockSpec((k, tn), lambda j: (0, j))],
        out_specs=pl.BlockSpec((m, tn), lambda j: (0, j)),
        compiler_params=_cparams(),
    )(a.astype(_BF), b.astype(_BF))


def kernel(stem1_w, stem1_scale, stem1_bias, stem2_w, stem2_scale, stem2_bias,
           layer3_w, layer3_scale, layer3_bias, layer4_w, layer4_scale,
           layer4_bias, aspp0_w, aspp0_scale, aspp0_bias, aspp1_w, aspp1_scale,
           aspp1_bias, aspp2_w, aspp2_scale, aspp2_bias, aspp3_w, aspp3_scale,
           aspp3_bias, aspp_pool_w, aspp_pool_scale, aspp_pool_bias,
           aspp_proj_w, aspp_proj_scale, aspp_proj_bias, dec_low_w,
           dec_low_scale, dec_low_bias, dec_conv1_w, dec_conv1_scale,
           dec_conv1_bias, dec_conv2_w, dec_conv2_scale, dec_conv2_bias,
           classifier_w, classifier_b, x):
    n, _, s, _ = x.shape
    xh = jnp.transpose(x, (0, 2, 3, 1)).astype(_BF).reshape(n, s, s * 3)
    sf, sd = s // 16, s // 4

    wf1 = _fold(dec_conv1_w, dec_conv1_scale)
    wbigs = _prep_weights([
        _conv_item(_fold(stem1_w, stem1_scale), s + 2, s // 2, 2, 1,
                   _rup((s + 2) * 3, 128)),
        _conv_item(_fold(stem2_w, stem2_scale), s // 2 + 2, s // 4, 2, 1,
                   _rup((s // 2 + 2) * 8, 128)),
        _conv_item(_fold(layer3_w, layer3_scale), s // 4 + 2, s // 8, 2, 1,
                   _rup((s // 4 + 2) * 16, 128)),
        _conv_item(_fold(layer4_w, layer4_scale), s // 8 + 2, sf, 2, 1,
                   _rup((s // 8 + 2) * 24, 128)),
        _conv_item(wf1[:, :, :16, :], sd + 2, sd, 1, 1,
                   _rup((sd + 2) * 16, 128)),
        _conv_item(wf1[:, :, 16:, :], sd + 2, sd, 1, 1,
                   _rup((sd + 2) * 8, 128)),
        _conv_item(_fold(dec_conv2_w, dec_conv2_scale), sd + 2, sd, 1, 1,
                   _rup((sd + 2) * 16, 128)),
        _conv_item(_fold(aspp1_w, aspp1_scale), sf + 12, sf, 1, 6,
                   (sf + 12) * 32),
        _cls_item(classifier_w.reshape(16, 21), sd),
    ])

    h1 = _s2conv(xh, wbigs[0], stem1_bias, s, 3, 8)
    h2 = _s2conv(h1, wbigs[1], stem2_bias, s // 2, 8, 16)
    h3 = _s2conv(h2, wbigs[2], layer3_bias, s // 4, 16, 24)
    h4 = _s2conv(h3, wbigs[3], layer4_bias, s // 8, 24, 32)

    wjf = _fold(aspp_proj_w, aspp_proj_scale).reshape(80, 16)
    ha = _aspp(
        h4,
        _fold(aspp0_w, aspp0_scale).reshape(32, 16),
        wbigs[7],
        _fold(aspp2_w[1:2, 1:2], aspp2_scale).reshape(32, 16),
        _fold(aspp3_w[1:2, 1:2], aspp3_scale).reshape(32, 16),
        _fold(aspp_pool_w, aspp_pool_scale).reshape(32, 16),
        [wjf[16 * i:16 * (i + 1), :] for i in range(5)],
        [aspp0_bias, aspp1_bias, aspp2_bias, aspp3_bias, aspp_pool_bias,
         aspp_proj_bias],
        sf, 32, 16)

    hu = _up832(ha, sf, sd, 16)
    lf = _flat1(h2, _fold(dec_low_w, dec_low_scale).reshape(16, 8),
                dec_low_bias, sd)
    d1 = _dec_conv([hu, lf], [16, 8], [wbigs[4], wbigs[5]],
                   dec_conv1_bias, sd, 16)
    nc = 21
    cls = _dec_conv([d1], [16], [wbigs[6]],
                    dec_conv2_bias, sd, 16,
                    chain_w=wbigs[8][0],
                    chain_b=jnp.repeat(classifier_b.astype(_F32),
                                       sd).reshape(1, -1))

    return _finup(cls, n, s, sd, nc)


# speedup vs baseline: 2.4445x; 1.1410x over previous
import functools

import jax
import jax.numpy as jnp
import numpy as np
from jax.experimental import pallas as pl
from jax.experimental.pallas import tpu as pltpu

_BF = jnp.bfloat16
_F32 = jnp.float32


def _rup(x, m):
    return ((x + m - 1) // m) * m


def _tile(m, target, align=8):
    t = min(target, m)
    t -= t % align
    while t >= align:
        if m % t == 0:
            return t
        t -= align
    return m


def _interp_mat(out_size, in_size):
    if out_size == 1 or in_size == 1:
        m = np.zeros((out_size, in_size), np.float32)
        m[:, 0] = 1.0
        return m
    src = np.arange(out_size, dtype=np.float64) * (in_size - 1) / (out_size - 1)
    i0 = np.clip(np.floor(src).astype(np.int64), 0, in_size - 1)
    i1 = np.clip(i0 + 1, 0, in_size - 1)
    w1 = (src - i0).astype(np.float32)
    w0 = 1.0 - w1
    m = np.zeros((out_size, in_size), np.float32)
    m[np.arange(out_size), i0] += w0
    m[np.arange(out_size), i1] += w1
    return m


def _cparams():
    return pltpu.CompilerParams(
        dimension_semantics=("parallel",),
        vmem_limit_bytes=64 * 1024 * 1024,
    )


def _kron_eye(w2d, blocks):
    k, n = w2d.shape
    mask = np.kron(np.eye(blocks, dtype=np.float32), np.ones((k, n), np.float32))
    return (jnp.asarray(mask)
            * jnp.tile(w2d.astype(_F32), (blocks, blocks))).astype(_BF)


def _fold(w, scale):
    wf = w.astype(_F32)
    if scale is not None:
        wf = wf * scale[None, None, None, :]
    return wf


def _btile(bias, blocks):
    return jnp.tile(bias.astype(_F32), blocks).reshape(1, -1)


def _prep_consts(wp, wo, cin, cout, stride, dil, kp):
    t1 = np.zeros((kp, cin), np.float32)
    t1[:wp * cin] = np.tile(np.eye(cin, dtype=np.float32), (wp, 1))
    t2 = np.tile(np.eye(cout, dtype=np.float32), (1, wo))
    masks = np.zeros((3, kp, wo * cout), np.float32)
    cols = np.arange(wo)
    for kw in range(3):
        msel = np.zeros((wp, wo), np.float32)
        msel[cols * stride + kw * dil, cols] = 1.0
        masks[kw, :wp * cin] = np.kron(msel, np.ones((cin, cout), np.float32))
    return (jnp.asarray(t1).astype(_BF), jnp.asarray(t2).astype(_BF),
            jnp.asarray(masks).astype(_BF))


def _prep_body(*refs, shapes):
    outs = refs[4 * len(shapes):]
    for idx, (ot, q) in enumerate(shapes):
        w_ref, t1_ref, t2_ref, m_ref = refs[4 * idx:4 * idx + 4]
        for kh in range(ot):
            acc = None
            for kw in range(q):
                wt = w_ref[q * kh + kw].astype(_BF)
                a = jnp.dot(t1_ref[...], wt, preferred_element_type=_F32)
                b = jnp.dot(a.astype(_BF), t2_ref[...],
                            preferred_element_type=_F32)
                term = b * m_ref[kw].astype(_F32)
                acc = term if acc is None else acc + term
            outs[idx][kh] = acc.astype(_BF)


def _prep_weights(items):
    ops, in_specs, out_shapes, out_specs, shapes = [], [], [], [], []
    for w_flat, t1, t2, masks, ot in items:
        ops += [w_flat, t1, t2, masks]
        in_specs += [pl.BlockSpec(w_flat.shape, lambda i: (0, 0, 0)),
                     pl.BlockSpec(t1.shape, lambda i: (0, 0)),
                     pl.BlockSpec(t2.shape, lambda i: (0, 0)),
                     pl.BlockSpec(masks.shape, lambda i: (0, 0, 0))]
        out_shapes.append(jax.ShapeDtypeStruct(
            (ot, masks.shape[1], masks.shape[2]), _BF))
        out_specs.append(pl.BlockSpec(
            (ot, masks.shape[1], masks.shape[2]), lambda i: (0, 0, 0)))
        shapes.append((ot, masks.shape[0]))
    return pl.pallas_call(
        functools.partial(_prep_body, shapes=tuple(shapes)),
        out_shape=tuple(out_shapes),
        grid=(1,),
        in_specs=in_specs,
        out_specs=tuple(out_specs),
        compiler_params=pltpu.CompilerParams(
            dimension_semantics=("arbitrary",),
            vmem_limit_bytes=64 * 1024 * 1024,
        ),
    )(*ops)


def _conv_item(wf, wp, wo, stride, dil, kp):
    cin, cout = wf.shape[2], wf.shape[3]
    t1, t2, masks = _prep_consts(wp, wo, cin, cout, stride, dil, kp)
    return (wf.reshape(9, cin, cout), t1, t2, masks, 3)


def _cls_item(wcls, wi):
    ci, nc = wcls.shape
    t1 = jnp.asarray(np.tile(np.eye(ci, dtype=np.float32), (wi, 1))).astype(_BF)
    t2 = jnp.asarray(np.kron(np.eye(nc, dtype=np.float32),
                             np.ones((1, wi), np.float32))).astype(_BF)
    mask = np.zeros((wi, ci, nc, wi), np.float32)
    for w in range(wi):
        mask[w, :, :, w] = 1.0
    masks = jnp.asarray(mask.reshape(1, wi * ci, nc * wi)).astype(_BF)
    return (wcls.astype(_F32).reshape(1, ci, nc), t1, t2, masks, 1)


def _s2conv_body(x_ref, w_ref, b_ref, o_ref, *, ho, kp):
    nb = o_ref.shape[0]
    xs = x_ref[...]
    acc = None
    for kh in range(3):
        if kh == 0:
            a = xs[:, 0:ho, 0:kp]
        elif kh == 1:
            a = xs[:, 0:ho, kp:2 * kp]
        else:
            a = xs[:, 1:ho + 1, 0:kp]
        d = jnp.dot(a.reshape(nb * ho, kp), w_ref[kh],
                    preferred_element_type=_F32)
        acc = d if acc is None else acc + d
    acc = jnp.maximum(acc + b_ref[...], 0.0)
    o_ref[...] = acc.reshape(nb, ho, acc.shape[-1]).astype(o_ref.dtype)


def _s2conv(x3, wbig, bias, wi, cin, cout):
    n, h, _ = x3.shape
    ho, wo = h // 2, wi // 2
    hp, wp = h + 2, wi + 2
    wpc = wp * cin
    kp = wbig.shape[1]
    xp = jnp.pad(x3, ((0, 0), (1, 1), (cin, kp - wpc + cin)))
    xp = xp.reshape(n, hp // 2, 2 * kp)
    bt = _btile(bias, wo)
    nb = min(max(256 // ho, 1), n)
    while n % nb:
        nb -= 1
    return pl.pallas_call(
        functools.partial(_s2conv_body, ho=ho, kp=kp),
        out_shape=jax.ShapeDtypeStruct((n, ho, wo * cout), _BF),
        grid=(n // nb,),
        in_specs=[pl.BlockSpec((nb, hp // 2, 2 * kp), lambda i: (i, 0, 0)),
                  pl.BlockSpec((3, kp, wo * cout), lambda i: (0, 0, 0)),
                  pl.BlockSpec((1, wo * cout), lambda i: (0, 0))],
        out_specs=pl.BlockSpec((nb, ho, wo * cout), lambda i: (i, 0, 0)),
        compiler_params=_cparams(),
    )(xp, wbig, bt)


def _s1pad(x3, wi, c):
    wpc = (wi + 2) * c
    kp = _rup(wpc, 128)
    return jnp.pad(x3, ((0, 0), (1, 1), (c, kp - wpc + c))), kp


def _dec_body(*refs, n_in, ho, kps, chain):
    x_refs = refs[:n_in]
    w_refs = refs[n_in:2 * n_in]
    b_ref = refs[2 * n_in]
    extra = refs[2 * n_in + 1:]
    nb = extra[-1].shape[0]
    acc = None
    for j in range(n_in):
        xs = x_refs[j][...]
        for kh in range(3):
            a = xs[:, kh:kh + ho, :].reshape(nb * ho, kps[j])
            d = jnp.dot(a, w_refs[j][kh], preferred_element_type=_F32)
            acc = d if acc is None else acc + d
    acc = jnp.maximum(acc + b_ref[...], 0.0)
    if chain:
        wc_ref, bc_ref, o_ref = extra
        acc2 = jnp.dot(acc.astype(_BF), wc_ref[...],
                       preferred_element_type=_F32) + bc_ref[...]
        o_ref[...] = acc2.reshape(nb, ho, acc2.shape[-1]).astype(o_ref.dtype)
    else:
        o_ref = extra[0]
        o_ref[...] = acc.reshape(nb, ho, acc.shape[-1]).astype(o_ref.dtype)


def _dec_conv(x3_list, cins, wbigs, bias, wi, cout, chain_w=None, chain_b=None):
    n, ho = x3_list[0].shape[0], x3_list[0].shape[1]
    xps, kps = [], []
    for x3, cin in zip(x3_list, cins):
        xp, kp = _s1pad(x3, wi, cin)
        xps.append(xp)
        kps.append(kp)
    bt = _btile(bias, wi)
    n_out = wi * cout
    chain = chain_w is not None
    if chain:
        wc, bc = chain_w, chain_b
        n_out = wc.shape[1]
    nb = min(max(256 // ho, 1), n)
    while n % nb:
        nb -= 1
    in_specs = (
        [pl.BlockSpec((nb, ho + 2, kp), lambda i: (i, 0, 0)) for kp in kps]
        + [pl.BlockSpec((3, kp, wi * cout), lambda i: (0, 0, 0)) for kp in kps]
        + [pl.BlockSpec((1, wi * cout), lambda i: (0, 0))]
    )
    ops = list(xps) + wbigs + [bt]
    if chain:
        in_specs += [pl.BlockSpec((wi * cout, n_out), lambda i: (0, 0)),
                     pl.BlockSpec((1, n_out), lambda i: (0, 0))]
        ops += [wc, bc]
    return pl.pallas_call(
        functools.partial(_dec_body, n_in=len(x3_list), ho=ho,
                          kps=tuple(kps), chain=chain),
        out_shape=jax.ShapeDtypeStruct((n, ho, n_out), _BF),
        grid=(n // nb,),
        in_specs=in_specs,
        out_specs=pl.BlockSpec((nb, ho, n_out), lambda i: (i, 0, 0)),
        compiler_params=_cparams(),
    )(*ops)


def _flat1_body(x_ref, w_ref, b_ref, o_ref):
    nb, ho, kp = x_ref.shape
    a = x_ref[...].reshape(nb * ho, kp)
    acc = jnp.maximum(jnp.dot(a, w_ref[...], preferred_element_type=_F32)
                      + b_ref[...], 0.0)
    o_ref[...] = acc.reshape(nb, ho, acc.shape[-1]).astype(o_ref.dtype)


def _flat1(x3, w2d, bias, wi):
    n, ho, _ = x3.shape
    wk = _kron_eye(w2d, wi)
    bt = _btile(bias, wi)
    n_out = wk.shape[1]
    nb = min(max(256 // ho, 1), n)
    while n % nb:
        nb -= 1
    return pl.pallas_call(
        _flat1_body,
        out_shape=jax.ShapeDtypeStruct((n, ho, n_out), _BF),
        grid=(n // nb,),
        in_specs=[pl.BlockSpec((nb, ho, x3.shape[2]), lambda i: (i, 0, 0)),
                  pl.BlockSpec((wk.shape[0], n_out), lambda i: (0, 0)),
                  pl.BlockSpec((1, n_out), lambda i: (0, 0))],
        out_specs=pl.BlockSpec((nb, ho, n_out), lambda i: (i, 0, 0)),
        compiler_params=_cparams(),
    )(x3, wk, bt)


def _aspp_body(h_ref, hp6_ref, w0_ref, w2_ref, w3_ref, wb1_ref, wp_ref,
               j0_ref, j1_ref, j2_ref, j3_ref, j4_ref, k8_ref,
               p2_ref, c8_ref, e2_ref,
               c0_ref, c1_ref, c2_ref, c3_ref, cp_ref, cj_ref, o_ref):
    nb, sf, lanes = o_ref.shape
    h = h_ref[...]
    b0 = jnp.maximum(jnp.dot(h, w0_ref[...], preferred_element_type=_F32)
                     + c0_ref[...], 0.0).astype(_BF)
    b2 = jnp.maximum(jnp.dot(h, w2_ref[...], preferred_element_type=_F32)
                     + c2_ref[...], 0.0).astype(_BF)
    b3 = jnp.maximum(jnp.dot(h, w3_ref[...], preferred_element_type=_F32)
                     + c3_ref[...], 0.0).astype(_BF)
    hp = hp6_ref[...]
    b1 = None
    for kh in range(3):
        a = hp[:, 6 * kh:6 * kh + sf, :].reshape(nb * sf, hp.shape[-1])
        d = jnp.dot(a, wb1_ref[kh], preferred_element_type=_F32)
        b1 = d if b1 is None else b1 + d
    b1 = jnp.maximum(b1 + c1_ref[...], 0.0).astype(_BF)
    acc = jnp.dot(b0, j0_ref[...], preferred_element_type=_F32)
    acc = acc + jnp.dot(b1, j1_ref[...], preferred_element_type=_F32)
    acc = acc + jnp.dot(b2, j2_ref[...], preferred_element_type=_F32)
    acc = acc + jnp.dot(b3, j3_ref[...], preferred_element_type=_F32)
    pr = jnp.dot(p2_ref[...], h, preferred_element_type=_F32)
    pm = jnp.dot(pr.astype(_BF), c8_ref[...], preferred_element_type=_F32)
    b4 = jnp.maximum(jnp.dot(pm.astype(_BF), wp_ref[...],
                             preferred_element_type=_F32) + cp_ref[...], 0.0)
    c4 = jnp.dot(b4.astype(_BF), j4_ref[...], preferred_element_type=_F32)
    c4t = jnp.dot(c4.astype(_BF), k8_ref[...], preferred_element_type=_F32)
    acc = acc + jnp.dot(e2_ref[...], c4t.astype(_BF),
                        preferred_element_type=_F32)
    acc = jnp.maximum(acc + cj_ref[...], 0.0)
    o_ref[...] = acc.reshape(nb, sf, lanes).astype(o_ref.dtype)


def _aspp(h4, w0, wb1, w2, w3, wp, wj, biases, sf, cm, co):
    n = h4.shape[0]
    hflat = h4.reshape(n * sf, sf * cm)
    hp6 = jnp.pad(h4, ((0, 0), (6, 6), (6 * cm, 6 * cm)))
    g = 2 if n % 2 == 0 else 1
    nb = n // g
    k8 = np.zeros((co * sf, co * sf), np.float32)
    for wi_ in range(sf):
        k8[0:co, wi_ * co:(wi_ + 1) * co] = np.eye(co)
    p2 = np.kron(np.eye(n, dtype=np.float32), np.full((1, sf), 1.0 / sf))
    c8 = np.kron(np.full((sf, 1), 1.0 / sf, np.float32), np.eye(cm))
    e2 = np.kron(np.eye(n, dtype=np.float32), np.ones((sf, 1), np.float32))
    c0, c1, c2, c3 = [_btile(b, sf) for b in biases[:4]]
    cp = jnp.pad(biases[4].astype(_F32).reshape(1, -1),
                 ((0, 0), (0, co * sf - co)))
    cj = _btile(biases[5], sf)
    jpads = [_kron_eye(w, sf) for w in wj[:4]] + [
        jnp.pad(wj[4].astype(_F32), ((0, co * sf - wj[4].shape[0]),
                                     (0, co * sf - wj[4].shape[1]))).astype(_BF)]
    wpp = jnp.pad(wp.astype(_F32), ((0, 0), (0, co * sf - co))).astype(_BF)
    lanes = sf * co
    in_specs = [
        pl.BlockSpec((nb * sf, sf * cm), lambda i: (i, 0)),
        pl.BlockSpec((nb, sf + 12, hp6.shape[2]), lambda i: (i, 0, 0)),
        pl.BlockSpec((sf * cm, lanes), lambda i: (0, 0)),
        pl.BlockSpec((sf * cm, lanes), lambda i: (0, 0)),
        pl.BlockSpec((sf * cm, lanes), lambda i: (0, 0)),
        pl.BlockSpec((3, hp6.shape[2], lanes), lambda i: (0, 0, 0)),
        pl.BlockSpec((cm, lanes), lambda i: (0, 0)),
    ] + [pl.BlockSpec((lanes, lanes), lambda i: (0, 0))] * 6 + [
        pl.BlockSpec((n, nb * sf), lambda i: (0, i)),
        pl.BlockSpec((sf * cm, cm), lambda i: (0, 0)),
        pl.BlockSpec((nb * sf, n), lambda i: (i, 0)),
    ] + [pl.BlockSpec((1, lanes), lambda i: (0, 0))] * 6
    return pl.pallas_call(
        _aspp_body,
        out_shape=jax.ShapeDtypeStruct((n, sf, lanes), _BF),
        grid=(g,),
        in_specs=in_specs,
        out_specs=pl.BlockSpec((nb, sf, lanes), lambda i: (i, 0, 0)),
        compiler_params=_cparams(),
    )(hflat, hp6,
      _kron_eye(w0, sf), _kron_eye(w2, sf), _kron_eye(w3, sf), wb1, wpp,
      *jpads, jnp.asarray(k8).astype(_BF),
      jnp.asarray(p2).astype(_BF), jnp.asarray(c8).astype(_BF),
      jnp.asarray(e2).astype(_BF),
      c0, c1, c2, c3, cp, cj)


def _up_body(x_ref, ww_ref, rh_ref, o_ref):
    nb, ho, lanes = o_ref.shape
    sf = x_ref.shape[1]
    xm = jnp.dot(x_ref[...].reshape(nb * sf, x_ref.shape[2]), ww_ref[...],
                 preferred_element_type=_F32)
    hu = jnp.dot(rh_ref[...], xm.astype(_BF), preferred_element_type=_F32)
    o_ref[...] = hu.reshape(nb, ho, lanes).astype(o_ref.dtype)


def _up832(x3, sf, sd, co):
    n = x3.shape[0]
    r1 = _interp_mat(sd, sf)
    ww = np.einsum('ow,ij->wioj', r1, np.eye(co, dtype=np.float32))
    ww = jnp.asarray(ww.reshape(sf * co, sd * co)).astype(_BF)
    bigrh = jnp.asarray(np.kron(np.eye(n, dtype=np.float32), r1)).astype(_BF)
    g = 2 if n % 2 == 0 else 1
    nb = n // g
    return pl.pallas_call(
        _up_body,
        out_shape=jax.ShapeDtypeStruct((n, sd, sd * co), _BF),
        grid=(g,),
        in_specs=[pl.BlockSpec((nb, sf, sf * co), lambda i: (i, 0, 0)),
                  pl.BlockSpec((sf * co, sd * co), lambda i: (0, 0)),
                  pl.BlockSpec((nb * sd, nb * sf), lambda i: (i, i))],
        out_specs=pl.BlockSpec((nb, sd, sd * co), lambda i: (i, 0, 0)),
        compiler_params=_cparams(),
    )(x3, ww, bigrh)


def _finup_body(c_ref, rhk_ref, rwt_ref, o_ref, *, nc, sd):
    nb, _, lanes = c_ref.shape
    s = o_ref.shape[2]
    cm = c_ref[...].reshape(nb * sd, lanes)
    t1 = jax.lax.dot_general(cm, rhk_ref[...], (((0,), (0,)), ((), ())),
                             preferred_element_type=_F32)
    t1 = t1.astype(_BF)
    for c in range(nc):
        tc = t1[sd * c:sd * (c + 1), :]
        oc = jax.lax.dot_general(tc, rwt_ref[...], (((0,), (0,)), ((), ())),
                                 preferred_element_type=_F32)
        o_ref[:, c, :, :] = oc.reshape(nb, s, s)


def _finup(cls3, n, s, sd, nc):
    nb = 4
    while n % nb:
        nb -= 1
    rh = _interp_mat(s, sd)
    rhk = jnp.asarray(np.kron(np.eye(nb, dtype=np.float32), rh.T)).astype(_BF)
    rwt = jnp.asarray(rh.T).astype(_BF)
    return pl.pallas_call(
        functools.partial(_finup_body, nc=nc, sd=sd),
        out_shape=jax.ShapeDtypeStruct((n, nc, s, s), _F32),
        grid=(n // nb,),
        in_specs=[pl.BlockSpec((nb, sd, nc * sd), lambda i: (i, 0, 0)),
                  pl.BlockSpec((nb * sd, nb * s), lambda i: (0, 0)),
                  pl.BlockSpec((sd, s), lambda i: (0, 0))],
        out_specs=pl.BlockSpec((nb, nc, s, s), lambda i: (i, 0, 0, 0)),
        compiler_params=_cparams(),
    )(cls3, rhk, rwt)


def _mm_body(a_ref, b_ref, o_ref):
    o_ref[...] = jnp.dot(a_ref[...], b_ref[...],
                         preferred_element_type=_F32).astype(o_ref.dtype)


def _mmT_body(a_ref, b_ref, o_ref):
    o_ref[...] = jax.lax.dot_general(
        a_ref[...], b_ref[...], (((0,), (0,)), ((), ())),
        preferred_element_type=_F32).astype(o_ref.dtype)


def _mmT(at, b, tile_m, out_dtype):
    k, m = at.shape
    n = b.shape[1]
    tm = _tile(m, tile_m, align=128)
    return pl.pallas_call(
        _mmT_body,
        out_shape=jax.ShapeDtypeStruct((m, n), out_dtype),
        grid=(m // tm,),
        in_specs=[pl.BlockSpec((k, tm), lambda i: (0, i)),
                  pl.BlockSpec((k, n), lambda i: (0, 0))],
        out_specs=pl.BlockSpec((tm, n), lambda i: (i, 0)),
        compiler_params=_cparams(),
    )(at.astype(_BF), b.astype(_BF))


def _col_mm(a, b, tile_n, out_dtype):
    m, k = a.shape
    n = b.shape[1]
    tn = _tile(n, tile_n, align=128)
    return pl.pallas_call(
        _mm_body,
        out_shape=jax.ShapeDtypeStruct((m, n), out_dtype),
        grid=(n // tn,),
        in_specs=[pl.BlockSpec((m, k), lambda j: (0, 0)),
                  pl.BlockSpec((k, tn), lambda j: (0, j))],
        out_specs=pl.BlockSpec((m, tn), lambda j: (0, j)),
        compiler_params=_cparams(),
    )(a.astype(_BF), b.astype(_BF))


def kernel(stem1_w, stem1_scale, stem1_bias, stem2_w, stem2_scale, stem2_bias,
           layer3_w, layer3_scale, layer3_bias, layer4_w, layer4_scale,
           layer4_bias, aspp0_w, aspp0_scale, aspp0_bias, aspp1_w, aspp1_scale,
           aspp1_bias, aspp2_w, aspp2_scale, aspp2_bias, aspp3_w, aspp3_scale,
           aspp3_bias, aspp_pool_w, aspp_pool_scale, aspp_pool_bias,
           aspp_proj_w, aspp_proj_scale, aspp_proj_bias, dec_low_w,
           dec_low_scale, dec_low_bias, dec_conv1_w, dec_conv1_scale,
           dec_conv1_bias, dec_conv2_w, dec_conv2_scale, dec_conv2_bias,
           classifier_w, classifier_b, x):
    n, _, s, _ = x.shape
    xh = jnp.transpose(x, (0, 2, 3, 1)).astype(_BF).reshape(n, s, s * 3)
    sf, sd = s // 16, s // 4

    wf1 = _fold(dec_conv1_w, dec_conv1_scale)
    wbigs = _prep_weights([
        _conv_item(_fold(stem1_w, stem1_scale), s + 2, s // 2, 2, 1,
                   _rup((s + 2) * 3, 128)),
        _conv_item(_fold(stem2_w, stem2_scale), s // 2 + 2, s // 4, 2, 1,
                   _rup((s // 2 + 2) * 8, 128)),
        _conv_item(_fold(layer3_w, layer3_scale), s // 4 + 2, s // 8, 2, 1,
                   _rup((s // 4 + 2) * 16, 128)),
        _conv_item(_fold(layer4_w, layer4_scale), s // 8 + 2, sf, 2, 1,
                   _rup((s // 8 + 2) * 24, 128)),
        _conv_item(wf1[:, :, :16, :], sd + 2, sd, 1, 1,
                   _rup((sd + 2) * 16, 128)),
        _conv_item(wf1[:, :, 16:, :], sd + 2, sd, 1, 1,
                   _rup((sd + 2) * 8, 128)),
        _conv_item(_fold(dec_conv2_w, dec_conv2_scale), sd + 2, sd, 1, 1,
                   _rup((sd + 2) * 16, 128)),
        _conv_item(_fold(aspp1_w, aspp1_scale), sf + 12, sf, 1, 6,
                   (sf + 12) * 32),
        _cls_item(classifier_w.reshape(16, 21), sd),
    ])

    h1 = _s2conv(xh, wbigs[0], stem1_bias, s, 3, 8)
    h2 = _s2conv(h1, wbigs[1], stem2_bias, s // 2, 8, 16)
    h3 = _s2conv(h2, wbigs[2], layer3_bias, s // 4, 16, 24)
    h4 = _s2conv(h3, wbigs[3], layer4_bias, s // 8, 24, 32)

    wjf = _fold(aspp_proj_w, aspp_proj_scale).reshape(80, 16)
    ha = _aspp(
        h4,
        _fold(aspp0_w, aspp0_scale).reshape(32, 16),
        wbigs[7],
        _fold(aspp2_w[1:2, 1:2], aspp2_scale).reshape(32, 16),
        _fold(aspp3_w[1:2, 1:2], aspp3_scale).reshape(32, 16),
        _fold(aspp_pool_w, aspp_pool_scale).reshape(32, 16),
        [wjf[16 * i:16 * (i + 1), :] for i in range(5)],
        [aspp0_bias, aspp1_bias, aspp2_bias, aspp3_bias, aspp_pool_bias,
         aspp_proj_bias],
        sf, 32, 16)

    hu = _up832(ha, sf, sd, 16)
    lf = _flat1(h2, _fold(dec_low_w, dec_low_scale).reshape(16, 8),
                dec_low_bias, sd)
    d1 = _dec_conv([hu, lf], [16, 8], [wbigs[4], wbigs[5]],
                   dec_conv1_bias, sd, 16)
    nc = 21
    cls = _dec_conv([d1], [16], [wbigs[6]],
                    dec_conv2_bias, sd, 16,
                    chain_w=wbigs[8][0],
                    chain_b=jnp.repeat(classifier_b.astype(_F32),
                                       sd).reshape(1, -1))

    return _finup(cls, n, s, sd, nc)


# speedup vs baseline: 2.5613x; 1.0478x over previous
import functools

import jax
import jax.numpy as jnp
import numpy as np
from jax.experimental import pallas as pl
from jax.experimental.pallas import tpu as pltpu

_BF = jnp.bfloat16
_F32 = jnp.float32


def _rup(x, m):
    return ((x + m - 1) // m) * m


def _tile(m, target, align=8):
    t = min(target, m)
    t -= t % align
    while t >= align:
        if m % t == 0:
            return t
        t -= align
    return m


def _interp_mat(out_size, in_size):
    if out_size == 1 or in_size == 1:
        m = np.zeros((out_size, in_size), np.float32)
        m[:, 0] = 1.0
        return m
    src = np.arange(out_size, dtype=np.float64) * (in_size - 1) / (out_size - 1)
    i0 = np.clip(np.floor(src).astype(np.int64), 0, in_size - 1)
    i1 = np.clip(i0 + 1, 0, in_size - 1)
    w1 = (src - i0).astype(np.float32)
    w0 = 1.0 - w1
    m = np.zeros((out_size, in_size), np.float32)
    m[np.arange(out_size), i0] += w0
    m[np.arange(out_size), i1] += w1
    return m


def _cparams():
    return pltpu.CompilerParams(
        dimension_semantics=("parallel",),
        vmem_limit_bytes=64 * 1024 * 1024,
    )


def _kron_eye(w2d, blocks):
    k, n = w2d.shape
    mask = np.kron(np.eye(blocks, dtype=np.float32), np.ones((k, n), np.float32))
    return (jnp.asarray(mask)
            * jnp.tile(w2d.astype(_F32), (blocks, blocks))).astype(_BF)


def _fold(w, scale):
    wf = w.astype(_F32)
    if scale is not None:
        wf = wf * scale[None, None, None, :]
    return wf


def _btile(bias, blocks):
    return jnp.tile(bias.astype(_F32), blocks).reshape(1, -1)


def _prep_consts(wp, wo, cin, cout, stride, dil, kp):
    t1 = np.zeros((kp, cin), np.float32)
    t1[:wp * cin] = np.tile(np.eye(cin, dtype=np.float32), (wp, 1))
    t2 = np.tile(np.eye(cout, dtype=np.float32), (1, wo))
    masks = np.zeros((3, kp, wo * cout), np.float32)
    cols = np.arange(wo)
    for kw in range(3):
        msel = np.zeros((wp, wo), np.float32)
        msel[cols * stride + kw * dil, cols] = 1.0
        masks[kw, :wp * cin] = np.kron(msel, np.ones((cin, cout), np.float32))
    return (jnp.asarray(t1).astype(_BF), jnp.asarray(t2).astype(_BF),
            jnp.asarray(masks).astype(_BF))


def _prep_body(*refs, shapes):
    outs = refs[4 * len(shapes):]
    for idx, (ot, q) in enumerate(shapes):
        w_ref, t1_ref, t2_ref, m_ref = refs[4 * idx:4 * idx + 4]
        for kh in range(ot):
            acc = None
            for kw in range(q):
                wt = w_ref[q * kh + kw].astype(_BF)
                a = jnp.dot(t1_ref[...], wt, preferred_element_type=_F32)
                b = jnp.dot(a.astype(_BF), t2_ref[...],
                            preferred_element_type=_F32)
                term = b * m_ref[kw].astype(_F32)
                acc = term if acc is None else acc + term
            outs[idx][kh] = acc.astype(_BF)


def _prep_weights(items):
    ops, in_specs, out_shapes, out_specs, shapes = [], [], [], [], []
    for w_flat, t1, t2, masks, ot in items:
        ops += [w_flat, t1, t2, masks]
        in_specs += [pl.BlockSpec(w_flat.shape, lambda i: (0, 0, 0)),
                     pl.BlockSpec(t1.shape, lambda i: (0, 0)),
                     pl.BlockSpec(t2.shape, lambda i: (0, 0)),
                     pl.BlockSpec(masks.shape, lambda i: (0, 0, 0))]
        out_shapes.append(jax.ShapeDtypeStruct(
            (ot, masks.shape[1], masks.shape[2]), _BF))
        out_specs.append(pl.BlockSpec(
            (ot, masks.shape[1], masks.shape[2]), lambda i: (0, 0, 0)))
        shapes.append((ot, masks.shape[0]))
    return pl.pallas_call(
        functools.partial(_prep_body, shapes=tuple(shapes)),
        out_shape=tuple(out_shapes),
        grid=(1,),
        in_specs=in_specs,
        out_specs=tuple(out_specs),
        compiler_params=pltpu.CompilerParams(
            dimension_semantics=("arbitrary",),
            vmem_limit_bytes=64 * 1024 * 1024,
        ),
    )(*ops)


def _conv_item(wf, wp, wo, stride, dil, kp):
    cin, cout = wf.shape[2], wf.shape[3]
    t1, t2, masks = _prep_consts(wp, wo, cin, cout, stride, dil, kp)
    return (wf.reshape(9, cin, cout), t1, t2, masks, 3)


def _cls_item(wcls, wi):
    ci, nc = wcls.shape
    t1 = jnp.asarray(np.tile(np.eye(ci, dtype=np.float32), (wi, 1))).astype(_BF)
    t2 = jnp.asarray(np.kron(np.eye(nc, dtype=np.float32),
                             np.ones((1, wi), np.float32))).astype(_BF)
    mask = np.zeros((wi, ci, nc, wi), np.float32)
    for w in range(wi):
        mask[w, :, :, w] = 1.0
    masks = jnp.asarray(mask.reshape(1, wi * ci, nc * wi)).astype(_BF)
    return (wcls.astype(_F32).reshape(1, ci, nc), t1, t2, masks, 1)


def _s2conv_body(x_ref, w_ref, b_ref, o_ref, *, ho, kp):
    nb = o_ref.shape[0]
    xs = x_ref[...]
    acc = None
    for kh in range(3):
        if kh == 0:
            a = xs[:, 0:ho, 0:kp]
        elif kh == 1:
            a = xs[:, 0:ho, kp:2 * kp]
        else:
            a = xs[:, 1:ho + 1, 0:kp]
        d = jnp.dot(a.reshape(nb * ho, kp), w_ref[kh],
                    preferred_element_type=_F32)
        acc = d if acc is None else acc + d
    acc = jnp.maximum(acc + b_ref[...], 0.0)
    o_ref[...] = acc.reshape(nb, ho, acc.shape[-1]).astype(o_ref.dtype)


def _s2conv(x3, wbig, bias, wi, cin, cout):
    n, h, _ = x3.shape
    ho, wo = h // 2, wi // 2
    hp, wp = h + 2, wi + 2
    wpc = wp * cin
    kp = wbig.shape[1]
    xp = jnp.pad(x3, ((0, 0), (1, 1), (cin, kp - wpc + cin)))
    xp = xp.reshape(n, hp // 2, 2 * kp)
    bt = _btile(bias, wo)
    nb = min(max(384 // ho, 1), n)
    while n % nb:
        nb -= 1
    return pl.pallas_call(
        functools.partial(_s2conv_body, ho=ho, kp=kp),
        out_shape=jax.ShapeDtypeStruct((n, ho, wo * cout), _BF),
        grid=(n // nb,),
        in_specs=[pl.BlockSpec((nb, hp // 2, 2 * kp), lambda i: (i, 0, 0)),
                  pl.BlockSpec((3, kp, wo * cout), lambda i: (0, 0, 0)),
                  pl.BlockSpec((1, wo * cout), lambda i: (0, 0))],
        out_specs=pl.BlockSpec((nb, ho, wo * cout), lambda i: (i, 0, 0)),
        compiler_params=_cparams(),
    )(xp, wbig, bt)


def _s1pad(x3, wi, c):
    wpc = (wi + 2) * c
    kp = _rup(wpc, 128)
    return jnp.pad(x3, ((0, 0), (1, 1), (c, kp - wpc + c))), kp


def _dec_body(*refs, n_in, ho, kps, chain):
    x_refs = refs[:n_in]
    w_refs = refs[n_in:2 * n_in]
    b_ref = refs[2 * n_in]
    extra = refs[2 * n_in + 1:]
    nb = extra[-1].shape[0]
    acc = None
    for j in range(n_in):
        xs = x_refs[j][...]
        for kh in range(3):
            a = xs[:, kh:kh + ho, :].reshape(nb * ho, kps[j])
            d = jnp.dot(a, w_refs[j][kh], preferred_element_type=_F32)
            acc = d if acc is None else acc + d
    acc = jnp.maximum(acc + b_ref[...], 0.0)
    if chain:
        wc_ref, bc_ref, o_ref = extra
        acc2 = jnp.dot(acc.astype(_BF), wc_ref[...],
                       preferred_element_type=_F32) + bc_ref[...]
        o_ref[...] = acc2.reshape(nb, ho, acc2.shape[-1]).astype(o_ref.dtype)
    else:
        o_ref = extra[0]
        o_ref[...] = acc.reshape(nb, ho, acc.shape[-1]).astype(o_ref.dtype)


def _dec_conv(x3_list, cins, wbigs, bias, wi, cout, chain_w=None, chain_b=None):
    n, ho = x3_list[0].shape[0], x3_list[0].shape[1]
    xps, kps = [], []
    for x3, cin in zip(x3_list, cins):
        xp, kp = _s1pad(x3, wi, cin)
        xps.append(xp)
        kps.append(kp)
    bt = _btile(bias, wi)
    n_out = wi * cout
    chain = chain_w is not None
    if chain:
        wc, bc = chain_w, chain_b
        n_out = wc.shape[1]
    nb = min(max(384 // ho, 1), n)
    while n % nb:
        nb -= 1
    in_specs = (
        [pl.BlockSpec((nb, ho + 2, kp), lambda i: (i, 0, 0)) for kp in kps]
        + [pl.BlockSpec((3, kp, wi * cout), lambda i: (0, 0, 0)) for kp in kps]
        + [pl.BlockSpec((1, wi * cout), lambda i: (0, 0))]
    )
    ops = list(xps) + wbigs + [bt]
    if chain:
        in_specs += [pl.BlockSpec((wi * cout, n_out), lambda i: (0, 0)),
                     pl.BlockSpec((1, n_out), lambda i: (0, 0))]
        ops += [wc, bc]
    return pl.pallas_call(
        functools.partial(_dec_body, n_in=len(x3_list), ho=ho,
                          kps=tuple(kps), chain=chain),
        out_shape=jax.ShapeDtypeStruct((n, ho, n_out), _BF),
        grid=(n // nb,),
        in_specs=in_specs,
        out_specs=pl.BlockSpec((nb, ho, n_out), lambda i: (i, 0, 0)),
        compiler_params=_cparams(),
    )(*ops)


def _flat1_body(x_ref, w_ref, b_ref, o_ref):
    nb, ho, kp = x_ref.shape
    a = x_ref[...].reshape(nb * ho, kp)
    acc = jnp.maximum(jnp.dot(a, w_ref[...], preferred_element_type=_F32)
                      + b_ref[...], 0.0)
    o_ref[...] = acc.reshape(nb, ho, acc.shape[-1]).astype(o_ref.dtype)


def _flat1(x3, w2d, bias, wi):
    n, ho, _ = x3.shape
    wk = _kron_eye(w2d, wi)
    bt = _btile(bias, wi)
    n_out = wk.shape[1]
    nb = min(max(384 // ho, 1), n)
    while n % nb:
        nb -= 1
    return pl.pallas_call(
        _flat1_body,
        out_shape=jax.ShapeDtypeStruct((n, ho, n_out), _BF),
        grid=(n // nb,),
        in_specs=[pl.BlockSpec((nb, ho, x3.shape[2]), lambda i: (i, 0, 0)),
                  pl.BlockSpec((wk.shape[0], n_out), lambda i: (0, 0)),
                  pl.BlockSpec((1, n_out), lambda i: (0, 0))],
        out_specs=pl.BlockSpec((nb, ho, n_out), lambda i: (i, 0, 0)),
        compiler_params=_cparams(),
    )(x3, wk, bt)


def _aspp_body(h_ref, hp6_ref, w0_ref, w2_ref, w3_ref, wb1_ref, wp_ref,
               j0_ref, j1_ref, j2_ref, j3_ref, j4_ref, k8_ref,
               p2_ref, c8_ref, e2_ref,
               c0_ref, c1_ref, c2_ref, c3_ref, cp_ref, cj_ref, o_ref):
    nb, sf, lanes = o_ref.shape
    h = h_ref[...]
    b0 = jnp.maximum(jnp.dot(h, w0_ref[...], preferred_element_type=_F32)
                     + c0_ref[...], 0.0).astype(_BF)
    b2 = jnp.maximum(jnp.dot(h, w2_ref[...], preferred_element_type=_F32)
                     + c2_ref[...], 0.0).astype(_BF)
    b3 = jnp.maximum(jnp.dot(h, w3_ref[...], preferred_element_type=_F32)
                     + c3_ref[...], 0.0).astype(_BF)
    hp = hp6_ref[...]
    b1 = None
    for kh in range(3):
        a = hp[:, 6 * kh:6 * kh + sf, :].reshape(nb * sf, hp.shape[-1])
        d = jnp.dot(a, wb1_ref[kh], preferred_element_type=_F32)
        b1 = d if b1 is None else b1 + d
    b1 = jnp.maximum(b1 + c1_ref[...], 0.0).astype(_BF)
    acc = jnp.dot(b0, j0_ref[...], preferred_element_type=_F32)
    acc = acc + jnp.dot(b1, j1_ref[...], preferred_element_type=_F32)
    acc = acc + jnp.dot(b2, j2_ref[...], preferred_element_type=_F32)
    acc = acc + jnp.dot(b3, j3_ref[...], preferred_element_type=_F32)
    pr = jnp.dot(p2_ref[...], h, preferred_element_type=_F32)
    pm = jnp.dot(pr.astype(_BF), c8_ref[...], preferred_element_type=_F32)
    b4 = jnp.maximum(jnp.dot(pm.astype(_BF), wp_ref[...],
                             preferred_element_type=_F32) + cp_ref[...], 0.0)
    c4 = jnp.dot(b4.astype(_BF), j4_ref[...], preferred_element_type=_F32)
    c4t = jnp.dot(c4.astype(_BF), k8_ref[...], preferred_element_type=_F32)
    acc = acc + jnp.dot(e2_ref[...], c4t.astype(_BF),
                        preferred_element_type=_F32)
    acc = jnp.maximum(acc + cj_ref[...], 0.0)
    o_ref[...] = acc.reshape(nb, sf, lanes).astype(o_ref.dtype)


def _aspp(h4, w0, wb1, w2, w3, wp, wj, biases, sf, cm, co):
    n = h4.shape[0]
    hflat = h4.reshape(n * sf, sf * cm)
    hp6 = jnp.pad(h4, ((0, 0), (6, 6), (6 * cm, 6 * cm)))
    g = 2 if n % 2 == 0 else 1
    nb = n // g
    k8 = np.zeros((co * sf, co * sf), np.float32)
    for wi_ in range(sf):
        k8[0:co, wi_ * co:(wi_ + 1) * co] = np.eye(co)
    p2 = np.kron(np.eye(n, dtype=np.float32), np.full((1, sf), 1.0 / sf))
    c8 = np.kron(np.full((sf, 1), 1.0 / sf, np.float32), np.eye(cm))
    e2 = np.kron(np.eye(n, dtype=np.float32), np.ones((sf, 1), np.float32))
    c0, c1, c2, c3 = [_btile(b, sf) for b in biases[:4]]
    cp = jnp.pad(biases[4].astype(_F32).reshape(1, -1),
                 ((0, 0), (0, co * sf - co)))
    cj = _btile(biases[5], sf)
    jpads = [_kron_eye(w, sf) for w in wj[:4]] + [
        jnp.pad(wj[4].astype(_F32), ((0, co * sf - wj[4].shape[0]),
                                     (0, co * sf - wj[4].shape[1]))).astype(_BF)]
    wpp = jnp.pad(wp.astype(_F32), ((0, 0), (0, co * sf - co))).astype(_BF)
    lanes = sf * co
    in_specs = [
        pl.BlockSpec((nb * sf, sf * cm), lambda i: (i, 0)),
        pl.BlockSpec((nb, sf + 12, hp6.shape[2]), lambda i: (i, 0, 0)),
        pl.BlockSpec((sf * cm, lanes), lambda i: (0, 0)),
        pl.BlockSpec((sf * cm, lanes), lambda i: (0, 0)),
        pl.BlockSpec((sf * cm, lanes), lambda i: (0, 0)),
        pl.BlockSpec((3, hp6.shape[2], lanes), lambda i: (0, 0, 0)),
        pl.BlockSpec((cm, lanes), lambda i: (0, 0)),
    ] + [pl.BlockSpec((lanes, lanes), lambda i: (0, 0))] * 6 + [
        pl.BlockSpec((n, nb * sf), lambda i: (0, i)),
        pl.BlockSpec((sf * cm, cm), lambda i: (0, 0)),
        pl.BlockSpec((nb * sf, n), lambda i: (i, 0)),
    ] + [pl.BlockSpec((1, lanes), lambda i: (0, 0))] * 6
    return pl.pallas_call(
        _aspp_body,
        out_shape=jax.ShapeDtypeStruct((n, sf, lanes), _BF),
        grid=(g,),
        in_specs=in_specs,
        out_specs=pl.BlockSpec((nb, sf, lanes), lambda i: (i, 0, 0)),
        compiler_params=_cparams(),
    )(hflat, hp6,
      _kron_eye(w0, sf), _kron_eye(w2, sf), _kron_eye(w3, sf), wb1, wpp,
      *jpads, jnp.asarray(k8).astype(_BF),
      jnp.asarray(p2).astype(_BF), jnp.asarray(c8).astype(_BF),
      jnp.asarray(e2).astype(_BF),
      c0, c1, c2, c3, cp, cj)


def _up_body(x_ref, ww_ref, rh_ref, o_ref):
    nb, ho, lanes = o_ref.shape
    sf = x_ref.shape[1]
    xm = jnp.dot(x_ref[...].reshape(nb * sf, x_ref.shape[2]), ww_ref[...],
                 preferred_element_type=_F32)
    hu = jnp.dot(rh_ref[...], xm.astype(_BF), preferred_element_type=_F32)
    o_ref[...] = hu.reshape(nb, ho, lanes).astype(o_ref.dtype)


def _up832(x3, sf, sd, co):
    n = x3.shape[0]
    r1 = _interp_mat(sd, sf)
    ww = np.einsum('ow,ij->wioj', r1, np.eye(co, dtype=np.float32))
    ww = jnp.asarray(ww.reshape(sf * co, sd * co)).astype(_BF)
    bigrh = jnp.asarray(np.kron(np.eye(n, dtype=np.float32), r1)).astype(_BF)
    g = 2 if n % 2 == 0 else 1
    nb = n // g
    return pl.pallas_call(
        _up_body,
        out_shape=jax.ShapeDtypeStruct((n, sd, sd * co), _BF),
        grid=(g,),
        in_specs=[pl.BlockSpec((nb, sf, sf * co), lambda i: (i, 0, 0)),
                  pl.BlockSpec((sf * co, sd * co), lambda i: (0, 0)),
                  pl.BlockSpec((nb * sd, nb * sf), lambda i: (i, i))],
        out_specs=pl.BlockSpec((nb, sd, sd * co), lambda i: (i, 0, 0)),
        compiler_params=_cparams(),
    )(x3, ww, bigrh)


def _finup_body(c_ref, rhk_ref, rwt_ref, o_ref, *, nc, sd):
    nb, _, lanes = c_ref.shape
    s = o_ref.shape[2]
    cm = c_ref[...].reshape(nb * sd, lanes)
    t1 = jax.lax.dot_general(cm, rhk_ref[...], (((0,), (0,)), ((), ())),
                             preferred_element_type=_F32)
    t1 = t1.astype(_BF)
    for c in range(nc):
        tc = t1[sd * c:sd * (c + 1), :]
        oc = jax.lax.dot_general(tc, rwt_ref[...], (((0,), (0,)), ((), ())),
                                 preferred_element_type=_F32)
        o_ref[:, c, :, :] = oc.reshape(nb, s, s)


def _finup(cls3, n, s, sd, nc):
    nb = 6
    while n % nb:
        nb -= 1
    rh = _interp_mat(s, sd)
    rhk = jnp.asarray(np.kron(np.eye(nb, dtype=np.float32), rh.T)).astype(_BF)
    rwt = jnp.asarray(rh.T).astype(_BF)
    return pl.pallas_call(
        functools.partial(_finup_body, nc=nc, sd=sd),
        out_shape=jax.ShapeDtypeStruct((n, nc, s, s), _F32),
        grid=(n // nb,),
        in_specs=[pl.BlockSpec((nb, sd, nc * sd), lambda i: (i, 0, 0)),
                  pl.BlockSpec((nb * sd, nb * s), lambda i: (0, 0)),
                  pl.BlockSpec((sd, s), lambda i: (0, 0))],
        out_specs=pl.BlockSpec((nb, nc, s, s), lambda i: (i, 0, 0, 0)),
        compiler_params=_cparams(),
    )(cls3, rhk, rwt)


def _mm_body(a_ref, b_ref, o_ref):
    o_ref[...] = jnp.dot(a_ref[...], b_ref[...],
                         preferred_element_type=_F32).astype(o_ref.dtype)


def _mmT_body(a_ref, b_ref, o_ref):
    o_ref[...] = jax.lax.dot_general(
        a_ref[...], b_ref[...], (((0,), (0,)), ((), ())),
        preferred_element_type=_F32).astype(o_ref.dtype)


def _mmT(at, b, tile_m, out_dtype):
    k, m = at.shape
    n = b.shape[1]
    tm = _tile(m, tile_m, align=128)
    return pl.pallas_call(
        _mmT_body,
        out_shape=jax.ShapeDtypeStruct((m, n), out_dtype),
        grid=(m // tm,),
        in_specs=[pl.BlockSpec((k, tm), lambda i: (0, i)),
                  pl.BlockSpec((k, n), lambda i: (0, 0))],
        out_specs=pl.BlockSpec((tm, n), lambda i: (i, 0)),
        compiler_params=_cparams(),
    )(at.astype(_BF), b.astype(_BF))


def _col_mm(a, b, tile_n, out_dtype):
    m, k = a.shape
    n = b.shape[1]
    tn = _tile(n, tile_n, align=128)
    return pl.pallas_call(
        _mm_body,
        out_shape=jax.ShapeDtypeStruct((m, n), out_dtype),
        grid=(n // tn,),
        in_specs=[pl.BlockSpec((m, k), lambda j: (0, 0)),
                  pl.BlockSpec((k, tn), lambda j: (0, j))],
        out_specs=pl.BlockSpec((m, tn), lambda j: (0, j)),
        compiler_params=_cparams(),
    )(a.astype(_BF), b.astype(_BF))


def kernel(stem1_w, stem1_scale, stem1_bias, stem2_w, stem2_scale, stem2_bias,
           layer3_w, layer3_scale, layer3_bias, layer4_w, layer4_scale,
           layer4_bias, aspp0_w, aspp0_scale, aspp0_bias, aspp1_w, aspp1_scale,
           aspp1_bias, aspp2_w, aspp2_scale, aspp2_bias, aspp3_w, aspp3_scale,
           aspp3_bias, aspp_pool_w, aspp_pool_scale, aspp_pool_bias,
           aspp_proj_w, aspp_proj_scale, aspp_proj_bias, dec_low_w,
           dec_low_scale, dec_low_bias, dec_conv1_w, dec_conv1_scale,
           dec_conv1_bias, dec_conv2_w, dec_conv2_scale, dec_conv2_bias,
           classifier_w, classifier_b, x):
    n, _, s, _ = x.shape
    xh = jnp.transpose(x, (0, 2, 3, 1)).astype(_BF).reshape(n, s, s * 3)
    sf, sd = s // 16, s // 4

    wf1 = _fold(dec_conv1_w, dec_conv1_scale)
    wbigs = _prep_weights([
        _conv_item(_fold(stem1_w, stem1_scale), s + 2, s // 2, 2, 1,
                   _rup((s + 2) * 3, 128)),
        _conv_item(_fold(stem2_w, stem2_scale), s // 2 + 2, s // 4, 2, 1,
                   _rup((s // 2 + 2) * 8, 128)),
        _conv_item(_fold(layer3_w, layer3_scale), s // 4 + 2, s // 8, 2, 1,
                   _rup((s // 4 + 2) * 16, 128)),
        _conv_item(_fold(layer4_w, layer4_scale), s // 8 + 2, sf, 2, 1,
                   _rup((s // 8 + 2) * 24, 128)),
        _conv_item(wf1[:, :, :16, :], sd + 2, sd, 1, 1,
                   _rup((sd + 2) * 16, 128)),
        _conv_item(wf1[:, :, 16:, :], sd + 2, sd, 1, 1,
                   _rup((sd + 2) * 8, 128)),
        _conv_item(_fold(dec_conv2_w, dec_conv2_scale), sd + 2, sd, 1, 1,
                   _rup((sd + 2) * 16, 128)),
        _conv_item(_fold(aspp1_w, aspp1_scale), sf + 12, sf, 1, 6,
                   (sf + 12) * 32),
        _cls_item(classifier_w.reshape(16, 21), sd),
    ])

    h1 = _s2conv(xh, wbigs[0], stem1_bias, s, 3, 8)
    h2 = _s2conv(h1, wbigs[1], stem2_bias, s // 2, 8, 16)
    h3 = _s2conv(h2, wbigs[2], layer3_bias, s // 4, 16, 24)
    h4 = _s2conv(h3, wbigs[3], layer4_bias, s // 8, 24, 32)

    wjf = _fold(aspp_proj_w, aspp_proj_scale).reshape(80, 16)
    ha = _aspp(
        h4,
        _fold(aspp0_w, aspp0_scale).reshape(32, 16),
        wbigs[7],
        _fold(aspp2_w[1:2, 1:2], aspp2_scale).reshape(32, 16),
        _fold(aspp3_w[1:2, 1:2], aspp3_scale).reshape(32, 16),
        _fold(aspp_pool_w, aspp_pool_scale).reshape(32, 16),
        [wjf[16 * i:16 * (i + 1), :] for i in range(5)],
        [aspp0_bias, aspp1_bias, aspp2_bias, aspp3_bias, aspp_pool_bias,
         aspp_proj_bias],
        sf, 32, 16)

    hu = _up832(ha, sf, sd, 16)
    lf = _flat1(h2, _fold(dec_low_w, dec_low_scale).reshape(16, 8),
                dec_low_bias, sd)
    d1 = _dec_conv([hu, lf], [16, 8], [wbigs[4], wbigs[5]],
                   dec_conv1_bias, sd, 16)
    nc = 21
    cls = _dec_conv([d1], [16], [wbigs[6]],
                    dec_conv2_bias, sd, 16,
                    chain_w=wbigs[8][0],
                    chain_b=jnp.repeat(classifier_b.astype(_F32),
                                       sd).reshape(1, -1))

    return _finup(cls, n, s, sd, nc)


# speedup vs baseline: 2.5980x; 1.0143x over previous
import functools

import jax
import jax.numpy as jnp
import numpy as np
from jax.experimental import pallas as pl
from jax.experimental.pallas import tpu as pltpu

_BF = jnp.bfloat16
_F32 = jnp.float32


def _rup(x, m):
    return ((x + m - 1) // m) * m


def _tile(m, target, align=8):
    t = min(target, m)
    t -= t % align
    while t >= align:
        if m % t == 0:
            return t
        t -= align
    return m


def _interp_mat(out_size, in_size):
    if out_size == 1 or in_size == 1:
        m = np.zeros((out_size, in_size), np.float32)
        m[:, 0] = 1.0
        return m
    src = np.arange(out_size, dtype=np.float64) * (in_size - 1) / (out_size - 1)
    i0 = np.clip(np.floor(src).astype(np.int64), 0, in_size - 1)
    i1 = np.clip(i0 + 1, 0, in_size - 1)
    w1 = (src - i0).astype(np.float32)
    w0 = 1.0 - w1
    m = np.zeros((out_size, in_size), np.float32)
    m[np.arange(out_size), i0] += w0
    m[np.arange(out_size), i1] += w1
    return m


def _cparams():
    return pltpu.CompilerParams(
        dimension_semantics=("parallel",),
        vmem_limit_bytes=64 * 1024 * 1024,
    )


def _kron_eye(w2d, blocks):
    k, n = w2d.shape
    mask = np.kron(np.eye(blocks, dtype=np.float32), np.ones((k, n), np.float32))
    return (jnp.asarray(mask)
            * jnp.tile(w2d.astype(_F32), (blocks, blocks))).astype(_BF)


def _fold(w, scale):
    wf = w.astype(_F32)
    if scale is not None:
        wf = wf * scale[None, None, None, :]
    return wf


def _btile(bias, blocks):
    return jnp.tile(bias.astype(_F32), blocks).reshape(1, -1)


def _prep_consts(wp, wo, cin, cout, stride, dil, kp):
    t1 = np.zeros((kp, cin), np.float32)
    t1[:wp * cin] = np.tile(np.eye(cin, dtype=np.float32), (wp, 1))
    t2 = np.tile(np.eye(cout, dtype=np.float32), (1, wo))
    masks = np.zeros((3, kp, wo * cout), np.float32)
    cols = np.arange(wo)
    for kw in range(3):
        msel = np.zeros((wp, wo), np.float32)
        msel[cols * stride + kw * dil, cols] = 1.0
        masks[kw, :wp * cin] = np.kron(msel, np.ones((cin, cout), np.float32))
    return (jnp.asarray(t1).astype(_BF), jnp.asarray(t2).astype(_BF),
            jnp.asarray(masks).astype(_BF))


def _prep_body(*refs, shapes):
    outs = refs[4 * len(shapes):]
    for idx, (ot, q) in enumerate(shapes):
        w_ref, t1_ref, t2_ref, m_ref = refs[4 * idx:4 * idx + 4]
        for kh in range(ot):
            acc = None
            for kw in range(q):
                wt = w_ref[q * kh + kw].astype(_BF)
                a = jnp.dot(t1_ref[...], wt, preferred_element_type=_F32)
                b = jnp.dot(a.astype(_BF), t2_ref[...],
                            preferred_element_type=_F32)
                term = b * m_ref[kw].astype(_F32)
                acc = term if acc is None else acc + term
            outs[idx][kh] = acc.astype(_BF)


def _prep_weights(items):
    ops, in_specs, out_shapes, out_specs, shapes = [], [], [], [], []
    for w_flat, t1, t2, masks, ot in items:
        ops += [w_flat, t1, t2, masks]
        in_specs += [pl.BlockSpec(w_flat.shape, lambda i: (0, 0, 0)),
                     pl.BlockSpec(t1.shape, lambda i: (0, 0)),
                     pl.BlockSpec(t2.shape, lambda i: (0, 0)),
                     pl.BlockSpec(masks.shape, lambda i: (0, 0, 0))]
        out_shapes.append(jax.ShapeDtypeStruct(
            (ot, masks.shape[1], masks.shape[2]), _BF))
        out_specs.append(pl.BlockSpec(
            (ot, masks.shape[1], masks.shape[2]), lambda i: (0, 0, 0)))
        shapes.append((ot, masks.shape[0]))
    return pl.pallas_call(
        functools.partial(_prep_body, shapes=tuple(shapes)),
        out_shape=tuple(out_shapes),
        grid=(1,),
        in_specs=in_specs,
        out_specs=tuple(out_specs),
        compiler_params=pltpu.CompilerParams(
            dimension_semantics=("arbitrary",),
            vmem_limit_bytes=64 * 1024 * 1024,
        ),
    )(*ops)


def _conv_item(wf, wp, wo, stride, dil, kp):
    cin, cout = wf.shape[2], wf.shape[3]
    t1, t2, masks = _prep_consts(wp, wo, cin, cout, stride, dil, kp)
    return (wf.reshape(9, cin, cout), t1, t2, masks, 3)


def _cls_item(wcls, wi):
    ci, nc = wcls.shape
    t1 = jnp.asarray(np.tile(np.eye(ci, dtype=np.float32), (wi, 1))).astype(_BF)
    t2 = jnp.asarray(np.kron(np.eye(nc, dtype=np.float32),
                             np.ones((1, wi), np.float32))).astype(_BF)
    mask = np.zeros((wi, ci, nc, wi), np.float32)
    for w in range(wi):
        mask[w, :, :, w] = 1.0
    masks = jnp.asarray(mask.reshape(1, wi * ci, nc * wi)).astype(_BF)
    return (wcls.astype(_F32).reshape(1, ci, nc), t1, t2, masks, 1)


def _s2conv_body(x_ref, w_ref, b_ref, o_ref, *, ho, kp):
    nb = o_ref.shape[0]
    xs = x_ref[...]
    acc = None
    for kh in range(3):
        if kh == 0:
            a = xs[:, 0:ho, 0:kp]
        elif kh == 1:
            a = xs[:, 0:ho, kp:2 * kp]
        else:
            a = xs[:, 1:ho + 1, 0:kp]
        d = jnp.dot(a.reshape(nb * ho, kp), w_ref[kh],
                    preferred_element_type=_F32)
        acc = d if acc is None else acc + d
    acc = jnp.maximum(acc + b_ref[...], 0.0)
    o_ref[...] = acc.reshape(nb, ho, acc.shape[-1]).astype(o_ref.dtype)


def _s2conv(x3, wbig, bias, wi, cin, cout):
    n, h, _ = x3.shape
    ho, wo = h // 2, wi // 2
    hp, wp = h + 2, wi + 2
    wpc = wp * cin
    kp = wbig.shape[1]
    xp = jnp.pad(x3, ((0, 0), (1, 1), (cin, kp - wpc + cin)))
    xp = xp.reshape(n, hp // 2, 2 * kp)
    bt = _btile(bias, wo)
    nb = min(max(512 // ho, 1), n)
    while n % nb:
        nb -= 1
    return pl.pallas_call(
        functools.partial(_s2conv_body, ho=ho, kp=kp),
        out_shape=jax.ShapeDtypeStruct((n, ho, wo * cout), _BF),
        grid=(n // nb,),
        in_specs=[pl.BlockSpec((nb, hp // 2, 2 * kp), lambda i: (i, 0, 0)),
                  pl.BlockSpec((3, kp, wo * cout), lambda i: (0, 0, 0)),
                  pl.BlockSpec((1, wo * cout), lambda i: (0, 0))],
        out_specs=pl.BlockSpec((nb, ho, wo * cout), lambda i: (i, 0, 0)),
        compiler_params=_cparams(),
    )(xp, wbig, bt)


def _s1pad(x3, wi, c):
    wpc = (wi + 2) * c
    kp = _rup(wpc, 128)
    return jnp.pad(x3, ((0, 0), (1, 1), (c, kp - wpc + c))), kp


def _dec_body(*refs, n_in, ho, kps, chain):
    x_refs = refs[:n_in]
    w_refs = refs[n_in:2 * n_in]
    b_ref = refs[2 * n_in]
    extra = refs[2 * n_in + 1:]
    nb = extra[-1].shape[0]
    acc = None
    for j in range(n_in):
        xs = x_refs[j][...]
        for kh in range(3):
            a = xs[:, kh:kh + ho, :].reshape(nb * ho, kps[j])
            d = jnp.dot(a, w_refs[j][kh], preferred_element_type=_F32)
            acc = d if acc is None else acc + d
    acc = jnp.maximum(acc + b_ref[...], 0.0)
    if chain:
        wc_ref, bc_ref, o_ref = extra
        acc2 = jnp.dot(acc.astype(_BF), wc_ref[...],
                       preferred_element_type=_F32) + bc_ref[...]
        o_ref[...] = acc2.reshape(nb, ho, acc2.shape[-1]).astype(o_ref.dtype)
    else:
        o_ref = extra[0]
        o_ref[...] = acc.reshape(nb, ho, acc.shape[-1]).astype(o_ref.dtype)


def _dec_conv(x3_list, cins, wbigs, bias, wi, cout, chain_w=None, chain_b=None):
    n, ho = x3_list[0].shape[0], x3_list[0].shape[1]
    xps, kps = [], []
    for x3, cin in zip(x3_list, cins):
        xp, kp = _s1pad(x3, wi, cin)
        xps.append(xp)
        kps.append(kp)
    bt = _btile(bias, wi)
    n_out = wi * cout
    chain = chain_w is not None
    if chain:
        wc, bc = chain_w, chain_b
        n_out = wc.shape[1]
    nb = min(max(512 // ho, 1), n)
    while n % nb:
        nb -= 1
    in_specs = (
        [pl.BlockSpec((nb, ho + 2, kp), lambda i: (i, 0, 0)) for kp in kps]
        + [pl.BlockSpec((3, kp, wi * cout), lambda i: (0, 0, 0)) for kp in kps]
        + [pl.BlockSpec((1, wi * cout), lambda i: (0, 0))]
    )
    ops = list(xps) + wbigs + [bt]
    if chain:
        in_specs += [pl.BlockSpec((wi * cout, n_out), lambda i: (0, 0)),
                     pl.BlockSpec((1, n_out), lambda i: (0, 0))]
        ops += [wc, bc]
    return pl.pallas_call(
        functools.partial(_dec_body, n_in=len(x3_list), ho=ho,
                          kps=tuple(kps), chain=chain),
        out_shape=jax.ShapeDtypeStruct((n, ho, n_out), _BF),
        grid=(n // nb,),
        in_specs=in_specs,
        out_specs=pl.BlockSpec((nb, ho, n_out), lambda i: (i, 0, 0)),
        compiler_params=_cparams(),
    )(*ops)


def _flat1_body(x_ref, w_ref, b_ref, o_ref):
    nb, ho, kp = x_ref.shape
    a = x_ref[...].reshape(nb * ho, kp)
    acc = jnp.maximum(jnp.dot(a, w_ref[...], preferred_element_type=_F32)
                      + b_ref[...], 0.0)
    o_ref[...] = acc.reshape(nb, ho, acc.shape[-1]).astype(o_ref.dtype)


def _flat1(x3, w2d, bias, wi):
    n, ho, _ = x3.shape
    wk = _kron_eye(w2d, wi)
    bt = _btile(bias, wi)
    n_out = wk.shape[1]
    nb = min(max(512 // ho, 1), n)
    while n % nb:
        nb -= 1
    return pl.pallas_call(
        _flat1_body,
        out_shape=jax.ShapeDtypeStruct((n, ho, n_out), _BF),
        grid=(n // nb,),
        in_specs=[pl.BlockSpec((nb, ho, x3.shape[2]), lambda i: (i, 0, 0)),
                  pl.BlockSpec((wk.shape[0], n_out), lambda i: (0, 0)),
                  pl.BlockSpec((1, n_out), lambda i: (0, 0))],
        out_specs=pl.BlockSpec((nb, ho, n_out), lambda i: (i, 0, 0)),
        compiler_params=_cparams(),
    )(x3, wk, bt)


def _aspp_body(h_ref, hp6_ref, w0_ref, w2_ref, w3_ref, wb1_ref, wp_ref,
               j0_ref, j1_ref, j2_ref, j3_ref, j4_ref, k8_ref,
               p2_ref, c8_ref, e2_ref,
               c0_ref, c1_ref, c2_ref, c3_ref, cp_ref, cj_ref, o_ref):
    nb, sf, lanes = o_ref.shape
    h = h_ref[...]
    b0 = jnp.maximum(jnp.dot(h, w0_ref[...], preferred_element_type=_F32)
                     + c0_ref[...], 0.0).astype(_BF)
    b2 = jnp.maximum(jnp.dot(h, w2_ref[...], preferred_element_type=_F32)
                     + c2_ref[...], 0.0).astype(_BF)
    b3 = jnp.maximum(jnp.dot(h, w3_ref[...], preferred_element_type=_F32)
                     + c3_ref[...], 0.0).astype(_BF)
    hp = hp6_ref[...]
    b1 = None
    for kh in range(3):
        a = hp[:, 6 * kh:6 * kh + sf, :].reshape(nb * sf, hp.shape[-1])
        d = jnp.dot(a, wb1_ref[kh], preferred_element_type=_F32)
        b1 = d if b1 is None else b1 + d
    b1 = jnp.maximum(b1 + c1_ref[...], 0.0).astype(_BF)
    acc = jnp.dot(b0, j0_ref[...], preferred_element_type=_F32)
    acc = acc + jnp.dot(b1, j1_ref[...], preferred_element_type=_F32)
    acc = acc + jnp.dot(b2, j2_ref[...], preferred_element_type=_F32)
    acc = acc + jnp.dot(b3, j3_ref[...], preferred_element_type=_F32)
    pr = jnp.dot(p2_ref[...], h, preferred_element_type=_F32)
    pm = jnp.dot(pr.astype(_BF), c8_ref[...], preferred_element_type=_F32)
    b4 = jnp.maximum(jnp.dot(pm.astype(_BF), wp_ref[...],
                             preferred_element_type=_F32) + cp_ref[...], 0.0)
    c4 = jnp.dot(b4.astype(_BF), j4_ref[...], preferred_element_type=_F32)
    c4t = jnp.dot(c4.astype(_BF), k8_ref[...], preferred_element_type=_F32)
    acc = acc + jnp.dot(e2_ref[...], c4t.astype(_BF),
                        preferred_element_type=_F32)
    acc = jnp.maximum(acc + cj_ref[...], 0.0)
    o_ref[...] = acc.reshape(nb, sf, lanes).astype(o_ref.dtype)


def _aspp(h4, w0, wb1, w2, w3, wp, wj, biases, sf, cm, co):
    n = h4.shape[0]
    hflat = h4.reshape(n * sf, sf * cm)
    hp6 = jnp.pad(h4, ((0, 0), (6, 6), (6 * cm, 6 * cm)))
    g = 2 if n % 2 == 0 else 1
    nb = n // g
    k8 = np.zeros((co * sf, co * sf), np.float32)
    for wi_ in range(sf):
        k8[0:co, wi_ * co:(wi_ + 1) * co] = np.eye(co)
    p2 = np.kron(np.eye(n, dtype=np.float32), np.full((1, sf), 1.0 / sf))
    c8 = np.kron(np.full((sf, 1), 1.0 / sf, np.float32), np.eye(cm))
    e2 = np.kron(np.eye(n, dtype=np.float32), np.ones((sf, 1), np.float32))
    c0, c1, c2, c3 = [_btile(b, sf) for b in biases[:4]]
    cp = jnp.pad(biases[4].astype(_F32).reshape(1, -1),
                 ((0, 0), (0, co * sf - co)))
    cj = _btile(biases[5], sf)
    jpads = [_kron_eye(w, sf) for w in wj[:4]] + [
        jnp.pad(wj[4].astype(_F32), ((0, co * sf - wj[4].shape[0]),
                                     (0, co * sf - wj[4].shape[1]))).astype(_BF)]
    wpp = jnp.pad(wp.astype(_F32), ((0, 0), (0, co * sf - co))).astype(_BF)
    lanes = sf * co
    in_specs = [
        pl.BlockSpec((nb * sf, sf * cm), lambda i: (i, 0)),
        pl.BlockSpec((nb, sf + 12, hp6.shape[2]), lambda i: (i, 0, 0)),
        pl.BlockSpec((sf * cm, lanes), lambda i: (0, 0)),
        pl.BlockSpec((sf * cm, lanes), lambda i: (0, 0)),
        pl.BlockSpec((sf * cm, lanes), lambda i: (0, 0)),
        pl.BlockSpec((3, hp6.shape[2], lanes), lambda i: (0, 0, 0)),
        pl.BlockSpec((cm, lanes), lambda i: (0, 0)),
    ] + [pl.BlockSpec((lanes, lanes), lambda i: (0, 0))] * 6 + [
        pl.BlockSpec((n, nb * sf), lambda i: (0, i)),
        pl.BlockSpec((sf * cm, cm), lambda i: (0, 0)),
        pl.BlockSpec((nb * sf, n), lambda i: (i, 0)),
    ] + [pl.BlockSpec((1, lanes), lambda i: (0, 0))] * 6
    return pl.pallas_call(
        _aspp_body,
        out_shape=jax.ShapeDtypeStruct((n, sf, lanes), _BF),
        grid=(g,),
        in_specs=in_specs,
        out_specs=pl.BlockSpec((nb, sf, lanes), lambda i: (i, 0, 0)),
        compiler_params=_cparams(),
    )(hflat, hp6,
      _kron_eye(w0, sf), _kron_eye(w2, sf), _kron_eye(w3, sf), wb1, wpp,
      *jpads, jnp.asarray(k8).astype(_BF),
      jnp.asarray(p2).astype(_BF), jnp.asarray(c8).astype(_BF),
      jnp.asarray(e2).astype(_BF),
      c0, c1, c2, c3, cp, cj)


def _up_body(x_ref, ww_ref, rh_ref, o_ref):
    nb, ho, lanes = o_ref.shape
    sf = x_ref.shape[1]
    xm = jnp.dot(x_ref[...].reshape(nb * sf, x_ref.shape[2]), ww_ref[...],
                 preferred_element_type=_F32)
    hu = jnp.dot(rh_ref[...], xm.astype(_BF), preferred_element_type=_F32)
    o_ref[...] = hu.reshape(nb, ho, lanes).astype(o_ref.dtype)


def _up832(x3, sf, sd, co):
    n = x3.shape[0]
    r1 = _interp_mat(sd, sf)
    ww = np.einsum('ow,ij->wioj', r1, np.eye(co, dtype=np.float32))
    ww = jnp.asarray(ww.reshape(sf * co, sd * co)).astype(_BF)
    bigrh = jnp.asarray(np.kron(np.eye(n, dtype=np.float32), r1)).astype(_BF)
    g = 2 if n % 2 == 0 else 1
    nb = n // g
    return pl.pallas_call(
        _up_body,
        out_shape=jax.ShapeDtypeStruct((n, sd, sd * co), _BF),
        grid=(g,),
        in_specs=[pl.BlockSpec((nb, sf, sf * co), lambda i: (i, 0, 0)),
                  pl.BlockSpec((sf * co, sd * co), lambda i: (0, 0)),
                  pl.BlockSpec((nb * sd, nb * sf), lambda i: (i, i))],
        out_specs=pl.BlockSpec((nb, sd, sd * co), lambda i: (i, 0, 0)),
        compiler_params=_cparams(),
    )(x3, ww, bigrh)


def _finup_body(c_ref, rhk_ref, rwt_ref, o_ref, *, nc, sd):
    nb, _, lanes = c_ref.shape
    s = o_ref.shape[2]
    cm = c_ref[...].reshape(nb * sd, lanes)
    t1 = jax.lax.dot_general(cm, rhk_ref[...], (((0,), (0,)), ((), ())),
                             preferred_element_type=_F32)
    t1 = t1.astype(_BF)
    for c in range(nc):
        tc = t1[sd * c:sd * (c + 1), :]
        oc = jax.lax.dot_general(tc, rwt_ref[...], (((0,), (0,)), ((), ())),
                                 preferred_element_type=_F32)
        o_ref[:, c, :, :] = oc.reshape(nb, s, s)


def _finup(cls3, n, s, sd, nc):
    nb = 6
    while n % nb:
        nb -= 1
    rh = _interp_mat(s, sd)
    rhk = jnp.asarray(np.kron(np.eye(nb, dtype=np.float32), rh.T)).astype(_BF)
    rwt = jnp.asarray(rh.T).astype(_BF)
    return pl.pallas_call(
        functools.partial(_finup_body, nc=nc, sd=sd),
        out_shape=jax.ShapeDtypeStruct((n, nc, s, s), _F32),
        grid=(n // nb,),
        in_specs=[pl.BlockSpec((nb, sd, nc * sd), lambda i: (i, 0, 0)),
                  pl.BlockSpec((nb * sd, nb * s), lambda i: (0, 0)),
                  pl.BlockSpec((sd, s), lambda i: (0, 0))],
        out_specs=pl.BlockSpec((nb, nc, s, s), lambda i: (i, 0, 0, 0)),
        compiler_params=_cparams(),
    )(cls3, rhk, rwt)


def _mm_body(a_ref, b_ref, o_ref):
    o_ref[...] = jnp.dot(a_ref[...], b_ref[...],
                         preferred_element_type=_F32).astype(o_ref.dtype)


def _mmT_body(a_ref, b_ref, o_ref):
    o_ref[...] = jax.lax.dot_general(
        a_ref[...], b_ref[...], (((0,), (0,)), ((), ())),
        preferred_element_type=_F32).astype(o_ref.dtype)


def _mmT(at, b, tile_m, out_dtype):
    k, m = at.shape
    n = b.shape[1]
    tm = _tile(m, tile_m, align=128)
    return pl.pallas_call(
        _mmT_body,
        out_shape=jax.ShapeDtypeStruct((m, n), out_dtype),
        grid=(m // tm,),
        in_specs=[pl.BlockSpec((k, tm), lambda i: (0, i)),
                  pl.BlockSpec((k, n), lambda i: (0, 0))],
        out_specs=pl.BlockSpec((tm, n), lambda i: (i, 0)),
        compiler_params=_cparams(),
    )(at.astype(_BF), b.astype(_BF))


def _col_mm(a, b, tile_n, out_dtype):
    m, k = a.shape
    n = b.shape[1]
    tn = _tile(n, tile_n, align=128)
    return pl.pallas_call(
        _mm_body,
        out_shape=jax.ShapeDtypeStruct((m, n), out_dtype),
        grid=(n // tn,),
        in_specs=[pl.BlockSpec((m, k), lambda j: (0, 0)),
                  pl.BlockSpec((k, tn), lambda j: (0, j))],
        out_specs=pl.BlockSpec((m, tn), lambda j: (0, j)),
        compiler_params=_cparams(),
    )(a.astype(_BF), b.astype(_BF))


def kernel(stem1_w, stem1_scale, stem1_bias, stem2_w, stem2_scale, stem2_bias,
           layer3_w, layer3_scale, layer3_bias, layer4_w, layer4_scale,
           layer4_bias, aspp0_w, aspp0_scale, aspp0_bias, aspp1_w, aspp1_scale,
           aspp1_bias, aspp2_w, aspp2_scale, aspp2_bias, aspp3_w, aspp3_scale,
           aspp3_bias, aspp_pool_w, aspp_pool_scale, aspp_pool_bias,
           aspp_proj_w, aspp_proj_scale, aspp_proj_bias, dec_low_w,
           dec_low_scale, dec_low_bias, dec_conv1_w, dec_conv1_scale,
           dec_conv1_bias, dec_conv2_w, dec_conv2_scale, dec_conv2_bias,
           classifier_w, classifier_b, x):
    n, _, s, _ = x.shape
    xh = jnp.transpose(x, (0, 2, 3, 1)).astype(_BF).reshape(n, s, s * 3)
    sf, sd = s // 16, s // 4

    wf1 = _fold(dec_conv1_w, dec_conv1_scale)
    wbigs = _prep_weights([
        _conv_item(_fold(stem1_w, stem1_scale), s + 2, s // 2, 2, 1,
                   _rup((s + 2) * 3, 128)),
        _conv_item(_fold(stem2_w, stem2_scale), s // 2 + 2, s // 4, 2, 1,
                   _rup((s // 2 + 2) * 8, 128)),
        _conv_item(_fold(layer3_w, layer3_scale), s // 4 + 2, s // 8, 2, 1,
                   _rup((s // 4 + 2) * 16, 128)),
        _conv_item(_fold(layer4_w, layer4_scale), s // 8 + 2, sf, 2, 1,
                   _rup((s // 8 + 2) * 24, 128)),
        _conv_item(wf1[:, :, :16, :], sd + 2, sd, 1, 1,
                   _rup((sd + 2) * 16, 128)),
        _conv_item(wf1[:, :, 16:, :], sd + 2, sd, 1, 1,
                   _rup((sd + 2) * 8, 128)),
        _conv_item(_fold(dec_conv2_w, dec_conv2_scale), sd + 2, sd, 1, 1,
                   _rup((sd + 2) * 16, 128)),
        _conv_item(_fold(aspp1_w, aspp1_scale), sf + 12, sf, 1, 6,
                   (sf + 12) * 32),
        _cls_item(classifier_w.reshape(16, 21), sd),
    ])

    h1 = _s2conv(xh, wbigs[0], stem1_bias, s, 3, 8)
    h2 = _s2conv(h1, wbigs[1], stem2_bias, s // 2, 8, 16)
    h3 = _s2conv(h2, wbigs[2], layer3_bias, s // 4, 16, 24)
    h4 = _s2conv(h3, wbigs[3], layer4_bias, s // 8, 24, 32)

    wjf = _fold(aspp_proj_w, aspp_proj_scale).reshape(80, 16)
    ha = _aspp(
        h4,
        _fold(aspp0_w, aspp0_scale).reshape(32, 16),
        wbigs[7],
        _fold(aspp2_w[1:2, 1:2], aspp2_scale).reshape(32, 16),
        _fold(aspp3_w[1:2, 1:2], aspp3_scale).reshape(32, 16),
        _fold(aspp_pool_w, aspp_pool_scale).reshape(32, 16),
        [wjf[16 * i:16 * (i + 1), :] for i in range(5)],
        [aspp0_bias, aspp1_bias, aspp2_bias, aspp3_bias, aspp_pool_bias,
         aspp_proj_bias],
        sf, 32, 16)

    hu = _up832(ha, sf, sd, 16)
    lf = _flat1(h2, _fold(dec_low_w, dec_low_scale).reshape(16, 8),
                dec_low_bias, sd)
    d1 = _dec_conv([hu, lf], [16, 8], [wbigs[4], wbigs[5]],
                   dec_conv1_bias, sd, 16)
    nc = 21
    cls = _dec_conv([d1], [16], [wbigs[6]],
                    dec_conv2_bias, sd, 16,
                    chain_w=wbigs[8][0],
                    chain_b=jnp.repeat(classifier_b.astype(_F32),
                                       sd).reshape(1, -1))

    return _finup(cls, n, s, sd, nc)
